# Initial kernel scaffold; baseline (speedup 1.0000x reference)
import jax
import jax.numpy as jnp
from jax.experimental import pallas as pl


def kernel(edge_index, edge_pairs, emb, W1, b1, W2, b2):
    raise NotImplementedError("write your pallas kernel here")


# speedup vs baseline: 15.7838x; 15.7838x over previous
import functools

import jax
import jax.numpy as jnp
from jax import lax
from jax.experimental import pallas as pl
from jax.experimental.pallas import tpu as pltpu
from jax.experimental.pallas import tpu_sc as plsc

N_NODES = 100000
P_NODES = 100352
DIM = 32
HALF = 16
N_EDGES = 1600000
EP = 1638400
EROWS = EP // 128
NW = 32
RW = EROWS // NW
SB = 16
NSB = RW // SB
ZROWS = P_NODES // 16
N_PAIRS = 262144
PROWS = N_PAIRS // 128
PR_W = PROWS // NW

_f32 = jnp.float32
_i32 = jnp.int32

_MESH = plsc.VectorSubcoreMesh(
    core_axis_name="c", subcore_axis_name="s", num_cores=2, num_subcores=16)


def _fill_rows(ref, nrows, value):
    def body(i, _):
        ref[i] = jnp.full((16,), value, _f32)
        return 0
    lax.fori_loop(0, nrows, body, 0)


@functools.partial(
    pl.kernel,
    out_type=jax.ShapeDtypeStruct((2, P_NODES, HALF), _f32),
    mesh=_MESH,
    compiler_params=pltpu.CompilerParams(use_tc_tiling_on_sc=False, needs_layout_passes=False),
    scratch_types=[
        pltpu.VMEM((SB, 128), _i32),
        pltpu.VMEM((128, HALF), _f32),
        pltpu.VMEM((128, HALF), _f32),
        pltpu.VMEM_SHARED((P_NODES, HALF), _f32),
    ],
)
def _hist_kernel(dst_hbm, out_hbm, dst_v, ones_v, zero_v, acc_sh):
    c = lax.axis_index("c")
    s = lax.axis_index("s")
    w = c * 16 + s
    _fill_rows(ones_v, 128, 1.0)
    _fill_rows(zero_v, 128, 0.0)

    def zero_body(i, _):
        pltpu.sync_copy(zero_v, acc_sh.at[pl.ds(s * ZROWS + i * 128, 128)])
        return 0
    lax.fori_loop(0, ZROWS // 128, zero_body, 0)
    plsc.subcore_barrier()

    def sblock(sb, _):
        pltpu.sync_copy(dst_hbm.at[pl.ds(w * RW + sb * SB, SB)], dst_v)

        def body(j, _):
            pltpu.sync_copy(ones_v, acc_sh.at[dst_v.at[j]], add=True)
            return 0
        lax.fori_loop(0, SB, body, 0)
        return 0
    lax.fori_loop(0, NSB, sblock, 0)
    plsc.subcore_barrier()
    pltpu.sync_copy(acc_sh.at[pl.ds(s * ZROWS, ZROWS)],
                    out_hbm.at[c, pl.ds(s * ZROWS, ZROWS)])


@functools.partial(
    pl.kernel,
    out_type=(jax.ShapeDtypeStruct((2, P_NODES, HALF), _f32),
              jax.ShapeDtypeStruct((2, P_NODES, HALF), _f32)),
    mesh=_MESH,
    compiler_params=pltpu.CompilerParams(use_tc_tiling_on_sc=False, needs_layout_passes=False),
    scratch_types=[
        pltpu.VMEM((SB, 128), _i32),
        pltpu.VMEM((SB, 128), _i32),
        pltpu.VMEM((128, HALF), _f32),
        pltpu.VMEM((128, HALF), _f32),
        pltpu.VMEM((128, HALF), _f32),
        pltpu.VMEM_SHARED((P_NODES, HALF), _f32),
        pltpu.SemaphoreType.DMA((2,)),
    ],
)
def _scatter_kernel(src_hbm, dst_hbm, glo_hbm, ghi_hbm, outlo_hbm, outhi_hbm,
                    src_v, dst_v, rows0_v, rows1_v, zero_v, acc_sh, sems):
    c = lax.axis_index("c")
    s = lax.axis_index("s")
    w = c * 16 + s
    _fill_rows(zero_v, 128, 0.0)

    def run_half(g_hbm, out_hbm):
        def zero_body(i, _):
            pltpu.sync_copy(zero_v, acc_sh.at[pl.ds(s * ZROWS + i * 128, 128)])
            return 0
        lax.fori_loop(0, ZROWS // 128, zero_body, 0)
        plsc.subcore_barrier()

        def gather(jj, rows, sem):
            return pltpu.make_async_copy(g_hbm.at[src_v.at[jj]], rows, sem)

        def sblock(sb, _):
            pltpu.sync_copy(src_hbm.at[pl.ds(w * RW + sb * SB, SB)], src_v)
            pltpu.sync_copy(dst_hbm.at[pl.ds(w * RW + sb * SB, SB)], dst_v)
            gather(0, rows0_v, sems.at[0]).start()

            def body(jp, _):
                j = jp * 2
                gather(j + 1, rows1_v, sems.at[1]).start()
                gather(j, rows0_v, sems.at[0]).wait()
                pltpu.sync_copy(rows0_v, acc_sh.at[dst_v.at[j]], add=True)

                @pl.when(j + 2 < SB)
                def _():
                    gather(j + 2, rows0_v, sems.at[0]).start()

                gather(j + 1, rows1_v, sems.at[1]).wait()
                pltpu.sync_copy(rows1_v, acc_sh.at[dst_v.at[j + 1]], add=True)
                return 0
            lax.fori_loop(0, SB // 2, body, 0)
            return 0
        lax.fori_loop(0, NSB, sblock, 0)
        plsc.subcore_barrier()
        pltpu.sync_copy(acc_sh.at[pl.ds(s * ZROWS, ZROWS)],
                        out_hbm.at[c, pl.ds(s * ZROWS, ZROWS)])
        plsc.subcore_barrier()

    run_half(glo_hbm, outlo_hbm)
    run_half(ghi_hbm, outhi_hbm)


@functools.partial(
    pl.kernel,
    out_type=jax.ShapeDtypeStruct((N_PAIRS,), _f32),
    mesh=_MESH,
    compiler_params=pltpu.CompilerParams(use_tc_tiling_on_sc=False, needs_layout_passes=False),
    scratch_types=[
        pltpu.VMEM((PR_W, 128), _i32),
        pltpu.VMEM((PR_W, 128), _i32),
        pltpu.VMEM((128, DIM), _f32),
        pltpu.VMEM((128, DIM), _f32),
        pltpu.VMEM((PR_W * 128,), _f32),
        pltpu.SemaphoreType.DMA((2,)),
    ],
)
def _decode_kernel(z_hbm, u_hbm, v_hbm, scores_hbm,
                   u_v, v_v, zu_v, zv_v, sc_v, sems):
    c = lax.axis_index("c")
    s = lax.axis_index("s")
    w = c * 16 + s
    pltpu.sync_copy(u_hbm.at[pl.ds(w * PR_W, PR_W)], u_v)
    pltpu.sync_copy(v_hbm.at[pl.ds(w * PR_W, PR_W)], v_v)
    iota16 = lax.iota(_i32, 16)

    def chunk(j, _):
        cu = pltpu.make_async_copy(z_hbm.at[u_v.at[j]], zu_v, sems.at[0])
        cv = pltpu.make_async_copy(z_hbm.at[v_v.at[j]], zv_v, sems.at[1])
        cu.start()
        cv.start()
        cu.wait()
        cv.wait()

        def grp(g, _):
            ridx = g * 16 + iota16

            def feat(jf, acc):
                cidx = jnp.full((16,), jf, _i32)
                return acc + (plsc.load_gather(zu_v, [ridx, cidx]) *
                              plsc.load_gather(zv_v, [ridx, cidx]))
            sc = lax.fori_loop(0, DIM, feat, jnp.zeros((16,), _f32))
            sc_v[pl.ds(j * 128 + g * 16, 16)] = sc
            return 0
        lax.fori_loop(0, 8, grp, 0)
        return 0
    lax.fori_loop(0, PR_W, chunk, 0)
    pltpu.sync_copy(sc_v, scores_hbm.at[pl.ds(w * PR_W * 128, PR_W * 128)])


_BLK = 2048
_GRID = P_NODES // _BLK


def _row_mask(i, x):
    rows = lax.broadcasted_iota(_i32, (x.shape[0], 1), 0) + i * _BLK
    return jnp.where(rows < N_NODES, x, 0.0)


def _tc_prep1(degp_ref, emb_ref, w1_ref, dinv_ref, glo_ref, ghi_ref):
    i = pl.program_id(0)
    deg = degp_ref[0] + degp_ref[1] + 1.0
    dinv = lax.rsqrt(deg)
    dinv_ref[...] = dinv
    h = jnp.dot(emb_ref[...], w1_ref[...], preferred_element_type=_f32)
    g = _row_mask(i, h * dinv[:, :1])
    glo_ref[...] = g[:, :HALF]
    ghi_ref[...] = g[:, HALF:]


def _tc_mid(alo_ref, ahi_ref, glo_ref, ghi_ref, dinv_ref, b1_ref, w2_ref,
            g2lo_ref, g2hi_ref):
    i = pl.program_id(0)
    dinv = dinv_ref[...]
    olo = dinv * (alo_ref[0] + alo_ref[1] + glo_ref[...]) + b1_ref[0, :HALF]
    ohi = dinv * (ahi_ref[0] + ahi_ref[1] + ghi_ref[...]) + b1_ref[0, HALF:]
    z1 = jax.nn.relu(jnp.concatenate([olo, ohi], axis=1))
    h = jnp.dot(z1, w2_ref[...], preferred_element_type=_f32)
    g = _row_mask(i, h * dinv[:, :1])
    g2lo_ref[...] = g[:, :HALF]
    g2hi_ref[...] = g[:, HALF:]


def _tc_final(alo_ref, ahi_ref, glo_ref, ghi_ref, dinv_ref, b2_ref, z_ref):
    dinv = dinv_ref[...]
    zlo = dinv * (alo_ref[0] + alo_ref[1] + glo_ref[...]) + b2_ref[0, :HALF]
    zhi = dinv * (ahi_ref[0] + ahi_ref[1] + ghi_ref[...]) + b2_ref[0, HALF:]
    z_ref[...] = jnp.concatenate([zlo, zhi], axis=1)


def _bs_half():
    return pl.BlockSpec((_BLK, HALF), lambda i: (i, 0))


def _bs_part():
    return pl.BlockSpec((2, _BLK, HALF), lambda i: (0, i, 0))


def _prep1(degp, emb_p, W1):
    return pl.pallas_call(
        _tc_prep1,
        grid=(_GRID,),
        in_specs=[_bs_part(),
                  pl.BlockSpec((_BLK, DIM), lambda i: (i, 0)),
                  pl.BlockSpec((DIM, DIM), lambda i: (0, 0))],
        out_specs=[_bs_half(), _bs_half(), _bs_half()],
        out_shape=[jax.ShapeDtypeStruct((P_NODES, HALF), _f32)] * 3,
    )(degp, emb_p, W1)


def _mid(alo, ahi, glo, ghi, dinv, b1, W2):
    return pl.pallas_call(
        _tc_mid,
        grid=(_GRID,),
        in_specs=[_bs_part(), _bs_part(), _bs_half(), _bs_half(), _bs_half(),
                  pl.BlockSpec((1, DIM), lambda i: (0, 0)),
                  pl.BlockSpec((DIM, DIM), lambda i: (0, 0))],
        out_specs=[_bs_half(), _bs_half()],
        out_shape=[jax.ShapeDtypeStruct((P_NODES, HALF), _f32)] * 2,
    )(alo, ahi, glo, ghi, dinv, b1, W2)


def _final(alo, ahi, glo, ghi, dinv, b2):
    return pl.pallas_call(
        _tc_final,
        grid=(_GRID,),
        in_specs=[_bs_part(), _bs_part(), _bs_half(), _bs_half(), _bs_half(),
                  pl.BlockSpec((1, DIM), lambda i: (0, 0))],
        out_specs=pl.BlockSpec((_BLK, DIM), lambda i: (i, 0)),
        out_shape=jax.ShapeDtypeStruct((P_NODES, DIM), _f32),
    )(alo, ahi, glo, ghi, dinv, b2)


def kernel(edge_index, edge_pairs, emb, W1, b1, W2, b2):
    pad = jnp.full((EP - N_EDGES,), N_NODES, _i32)
    src2d = jnp.concatenate([edge_index[0], pad]).reshape(EROWS, 128)
    dst2d = jnp.concatenate([edge_index[1], pad]).reshape(EROWS, 128)
    u2d = edge_pairs[0].reshape(PROWS, 128)
    v2d = edge_pairs[1].reshape(PROWS, 128)
    emb_p = jnp.pad(emb, ((0, P_NODES - N_NODES), (0, 0)))
    b1r = b1.reshape(1, DIM)
    b2r = b2.reshape(1, DIM)

    degp = _hist_kernel(dst2d)
    dinv, g1lo, g1hi = _prep1(degp, emb_p, W1)
    a1lo, a1hi = _scatter_kernel(src2d, dst2d, g1lo, g1hi)
    g2lo, g2hi = _mid(a1lo, a1hi, g1lo, g1hi, dinv, b1r, W2)
    a2lo, a2hi = _scatter_kernel(src2d, dst2d, g2lo, g2hi)
    z = _final(a2lo, a2hi, g2lo, g2hi, dinv, b2r)
    return _decode_kernel(z, u2d, v2d)


# speedup vs baseline: 16.3918x; 1.0385x over previous
import functools

import jax
import jax.numpy as jnp
from jax import lax
from jax.experimental import pallas as pl
from jax.experimental.pallas import tpu as pltpu
from jax.experimental.pallas import tpu_sc as plsc

N_NODES = 100000
P_NODES = 100352
DIM = 32
HALF = 16
N_EDGES = 1600000
EP = 1638400
EROWS = EP // 128
NW = 32
RW = EROWS // NW
SB = 16
NSB = RW // SB
CH = 512
EW = EP // NW
ZROWS = P_NODES // 16
N_PAIRS = 262144
PROWS = N_PAIRS // 128
PR_W = PROWS // NW

_f32 = jnp.float32
_i32 = jnp.int32

_MESH = plsc.VectorSubcoreMesh(
    core_axis_name="c", subcore_axis_name="s", num_cores=2, num_subcores=16)


def _fill_rows(ref, nrows, value):
    def body(i, _):
        ref[i] = jnp.full((16,), value, _f32)
        return 0
    lax.fori_loop(0, nrows, body, 0)


@functools.partial(
    pl.kernel,
    out_type=jax.ShapeDtypeStruct((2, P_NODES, HALF), _f32),
    mesh=_MESH,
    compiler_params=pltpu.CompilerParams(use_tc_tiling_on_sc=False, needs_layout_passes=False),
    scratch_types=[
        pltpu.VMEM((SB, 128), _i32),
        pltpu.VMEM((SB, 128), _i32),
        pltpu.VMEM((128, HALF), _f32),
        pltpu.VMEM((128, HALF), _f32),
        pltpu.VMEM_SHARED((P_NODES, HALF), _f32),
        pltpu.SemaphoreType.DMA((2,)),
        pltpu.SemaphoreType.DMA,
    ],
)
def _hist_kernel(dst_hbm, out_hbm, dst0_v, dst1_v, ones_v, zero_v, acc_sh,
                 ssems, zsem):
    c = lax.axis_index("c")
    s = lax.axis_index("s")
    w = c * 16 + s
    _fill_rows(ones_v, 128, 1.0)
    _fill_rows(zero_v, 128, 0.0)
    dbufs = (dst0_v, dst1_v)

    def zfire(i, _):
        pltpu.make_async_copy(
            zero_v, acc_sh.at[pl.ds(s * ZROWS + i * 128, 128)], zsem).start()
        return 0
    lax.fori_loop(0, ZROWS // 128, zfire, 0)

    def zdrain(i, _):
        pltpu.make_async_copy(
            zero_v, acc_sh.at[pl.ds(s * ZROWS, 128)], zsem).wait()
        return 0
    lax.fori_loop(0, ZROWS // 128, zdrain, 0)
    plsc.subcore_barrier()

    def stage(sb, buf):
        pltpu.sync_copy(dst_hbm.at[pl.ds(w * RW + sb * SB, SB)], buf)

    def fire(buf, sem):
        for j in range(SB):
            pltpu.async_copy(ones_v, acc_sh.at[buf.at[j]], sem, add=True)

    def drain(buf, sem):
        for j in range(SB):
            pltpu.make_async_copy(ones_v, acc_sh.at[buf.at[j]], sem).wait()

    stage(0, dst0_v)
    fire(dst0_v, ssems.at[0])

    def sblock(p, _):
        sb = p * 2

        @pl.when(sb + 1 < NSB)
        def _():
            stage(sb + 1, dst1_v)
            fire(dst1_v, ssems.at[1])
        drain(dst0_v, ssems.at[0])

        @pl.when(sb + 2 < NSB)
        def _():
            stage(sb + 2, dst0_v)
            fire(dst0_v, ssems.at[0])

        @pl.when(sb + 1 < NSB)
        def _():
            drain(dst1_v, ssems.at[1])
        return 0
    lax.fori_loop(0, (NSB + 1) // 2, sblock, 0)
    plsc.subcore_barrier()
    pltpu.sync_copy(acc_sh.at[pl.ds(s * ZROWS, ZROWS)],
                    out_hbm.at[c, pl.ds(s * ZROWS, ZROWS)])


@functools.partial(
    pl.kernel,
    out_type=(jax.ShapeDtypeStruct((2, P_NODES, HALF), _f32),
              jax.ShapeDtypeStruct((2, P_NODES, HALF), _f32)),
    mesh=_MESH,
    compiler_params=pltpu.CompilerParams(use_tc_tiling_on_sc=False, needs_layout_passes=False),
    scratch_types=[
        pltpu.VMEM((4 * CH,), _i32),
        pltpu.VMEM((SB, 128), _i32),
        pltpu.VMEM((CH, HALF), _f32),
        pltpu.VMEM((CH, HALF), _f32),
        pltpu.VMEM((128, HALF), _f32),
        pltpu.VMEM_SHARED((P_NODES, HALF), _f32),
        pltpu.SemaphoreType.DMA((2,)),
        pltpu.SemaphoreType.DMA((2,)),
        pltpu.SemaphoreType.DMA,
    ],
)
def _scatter_kernel(srcf_hbm, dst_hbm, glo_hbm, ghi_hbm, outlo_hbm, outhi_hbm,
                    src_v, dst_v, rb0_v, rb1_v, zero_v, acc_sh,
                    gsems, ssems, zsem):
    c = lax.axis_index("c")
    s = lax.axis_index("s")
    w = c * 16 + s
    _fill_rows(zero_v, 128, 0.0)
    rbufs = (rb0_v, rb1_v)

    def run_half(g_hbm, out_hbm):
        def zfire(i, _):
            pltpu.make_async_copy(
                zero_v, acc_sh.at[pl.ds(s * ZROWS + i * 128, 128)],
                zsem).start()
            return 0
        lax.fori_loop(0, ZROWS // 128, zfire, 0)

        def zdrain(i, _):
            pltpu.make_async_copy(
                zero_v, acc_sh.at[pl.ds(s * ZROWS, 128)], zsem).wait()
            return 0
        lax.fori_loop(0, ZROWS // 128, zdrain, 0)
        plsc.subcore_barrier()

        def gather(k, buf, sem):
            return pltpu.make_async_copy(
                g_hbm.at[src_v.at[pl.ds(k * CH, CH)]], buf, sem)

        def scat_start(k, q, sem):
            buf = rbufs[k % 2]
            pltpu.async_copy(buf.at[pl.ds(q * 128, 128)],
                             acc_sh.at[dst_v.at[k * 4 + q]], sem, add=True)

        def scat_wait(k, q, sem):
            buf = rbufs[k % 2]
            pltpu.make_async_copy(buf.at[pl.ds(q * 128, 128)],
                                  acc_sh.at[dst_v.at[k * 4 + q]], sem).wait()

        def sblock(sb, _):
            pltpu.sync_copy(srcf_hbm.at[pl.ds(w * EW + sb * 4 * CH, 4 * CH)],
                            src_v)
            pltpu.sync_copy(dst_hbm.at[pl.ds(w * RW + sb * SB, SB)], dst_v)
            gather(0, rb0_v, gsems.at[0]).start()
            gather(1, rb1_v, gsems.at[1]).start()
            for k in range(4):
                sl = k % 2
                gather(k, rbufs[sl], gsems.at[sl]).wait()
                for q in range(4):
                    scat_start(k, q, ssems.at[sl])
                if k + 2 < 4:
                    for q in range(4):
                        scat_wait(k, q, ssems.at[sl])
                    gather(k + 2, rbufs[sl], gsems.at[sl]).start()
            for k in (2, 3):
                for q in range(4):
                    scat_wait(k, q, ssems.at[k % 2])
            return 0
        lax.fori_loop(0, EW // (4 * CH), sblock, 0)
        plsc.subcore_barrier()
        pltpu.sync_copy(acc_sh.at[pl.ds(s * ZROWS, ZROWS)],
                        out_hbm.at[c, pl.ds(s * ZROWS, ZROWS)])
        plsc.subcore_barrier()

    run_half(glo_hbm, outlo_hbm)
    run_half(ghi_hbm, outhi_hbm)


@functools.partial(
    pl.kernel,
    out_type=jax.ShapeDtypeStruct((N_PAIRS,), _f32),
    mesh=_MESH,
    compiler_params=pltpu.CompilerParams(use_tc_tiling_on_sc=False, needs_layout_passes=False),
    scratch_types=[
        pltpu.VMEM((PR_W, 128), _i32),
        pltpu.VMEM((PR_W, 128), _i32),
        pltpu.VMEM((128, DIM), _f32),
        pltpu.VMEM((128, DIM), _f32),
        pltpu.VMEM((128, DIM), _f32),
        pltpu.VMEM((128, DIM), _f32),
        pltpu.VMEM((PR_W * 128,), _f32),
        pltpu.SemaphoreType.DMA((2,)),
        pltpu.SemaphoreType.DMA((2,)),
    ],
)
def _decode_kernel(z_hbm, u_hbm, v_hbm, scores_hbm,
                   u_v, v_v, zu0_v, zv0_v, zu1_v, zv1_v, sc_v, usems, vsems):
    c = lax.axis_index("c")
    s = lax.axis_index("s")
    w = c * 16 + s
    pltpu.sync_copy(u_hbm.at[pl.ds(w * PR_W, PR_W)], u_v)
    pltpu.sync_copy(v_hbm.at[pl.ds(w * PR_W, PR_W)], v_v)
    iota16 = lax.iota(_i32, 16)
    zubufs = (zu0_v, zu1_v)
    zvbufs = (zv0_v, zv1_v)

    def fetch(j, sl):
        return (pltpu.make_async_copy(z_hbm.at[u_v.at[j]], zubufs[sl],
                                      usems.at[sl]),
                pltpu.make_async_copy(z_hbm.at[v_v.at[j]], zvbufs[sl],
                                      vsems.at[sl]))

    def compute(j, sl):
        zu, zv = zubufs[sl], zvbufs[sl]

        def grp(g, _):
            ridx = g * 16 + iota16

            def feat(jf, acc):
                cidx = jnp.full((16,), jf, _i32)
                return acc + (plsc.load_gather(zu, [ridx, cidx]) *
                              plsc.load_gather(zv, [ridx, cidx]))
            sc = lax.fori_loop(0, DIM, feat, jnp.zeros((16,), _f32))
            sc_v[pl.ds(j * 128 + g * 16, 16)] = sc
            return 0
        lax.fori_loop(0, 8, grp, 0)

    for desc in fetch(0, 0):
        desc.start()
    for desc in fetch(1, 1):
        desc.start()

    def chunk(jp, _):
        j = jp * 2
        for desc in fetch(j, 0):
            desc.wait()
        compute(j, 0)

        @pl.when(j + 2 < PR_W)
        def _():
            for desc in fetch(j + 2, 0):
                desc.start()

        for desc in fetch(j + 1, 1):
            desc.wait()
        compute(j + 1, 1)

        @pl.when(j + 3 < PR_W)
        def _():
            for desc in fetch(j + 3, 1):
                desc.start()
        return 0
    lax.fori_loop(0, PR_W // 2, chunk, 0)
    pltpu.sync_copy(sc_v, scores_hbm.at[pl.ds(w * PR_W * 128, PR_W * 128)])


_BLK = 2048
_GRID = P_NODES // _BLK


def _row_mask(i, x):
    rows = lax.broadcasted_iota(_i32, (x.shape[0], 1), 0) + i * _BLK
    return jnp.where(rows < N_NODES, x, 0.0)


def _tc_prep1(degp_ref, emb_ref, w1_ref, dinv_ref, glo_ref, ghi_ref):
    i = pl.program_id(0)
    deg = degp_ref[0] + degp_ref[1] + 1.0
    dinv = lax.rsqrt(deg)
    dinv_ref[...] = dinv
    h = jnp.dot(emb_ref[...], w1_ref[...], preferred_element_type=_f32)
    g = _row_mask(i, h * dinv[:, :1])
    glo_ref[...] = g[:, :HALF]
    ghi_ref[...] = g[:, HALF:]


def _tc_mid(alo_ref, ahi_ref, glo_ref, ghi_ref, dinv_ref, b1_ref, w2_ref,
            g2lo_ref, g2hi_ref):
    i = pl.program_id(0)
    dinv = dinv_ref[...]
    olo = dinv * (alo_ref[0] + alo_ref[1] + glo_ref[...]) + b1_ref[0, :HALF]
    ohi = dinv * (ahi_ref[0] + ahi_ref[1] + ghi_ref[...]) + b1_ref[0, HALF:]
    z1 = jax.nn.relu(jnp.concatenate([olo, ohi], axis=1))
    h = jnp.dot(z1, w2_ref[...], preferred_element_type=_f32)
    g = _row_mask(i, h * dinv[:, :1])
    g2lo_ref[...] = g[:, :HALF]
    g2hi_ref[...] = g[:, HALF:]


def _tc_final(alo_ref, ahi_ref, glo_ref, ghi_ref, dinv_ref, b2_ref, z_ref):
    dinv = dinv_ref[...]
    zlo = dinv * (alo_ref[0] + alo_ref[1] + glo_ref[...]) + b2_ref[0, :HALF]
    zhi = dinv * (ahi_ref[0] + ahi_ref[1] + ghi_ref[...]) + b2_ref[0, HALF:]
    z_ref[...] = jnp.concatenate([zlo, zhi], axis=1)


def _bs_half():
    return pl.BlockSpec((_BLK, HALF), lambda i: (i, 0))


def _bs_part():
    return pl.BlockSpec((2, _BLK, HALF), lambda i: (0, i, 0))


def _prep1(degp, emb_p, W1):
    return pl.pallas_call(
        _tc_prep1,
        grid=(_GRID,),
        in_specs=[_bs_part(),
                  pl.BlockSpec((_BLK, DIM), lambda i: (i, 0)),
                  pl.BlockSpec((DIM, DIM), lambda i: (0, 0))],
        out_specs=[_bs_half(), _bs_half(), _bs_half()],
        out_shape=[jax.ShapeDtypeStruct((P_NODES, HALF), _f32)] * 3,
    )(degp, emb_p, W1)


def _mid(alo, ahi, glo, ghi, dinv, b1, W2):
    return pl.pallas_call(
        _tc_mid,
        grid=(_GRID,),
        in_specs=[_bs_part(), _bs_part(), _bs_half(), _bs_half(), _bs_half(),
                  pl.BlockSpec((1, DIM), lambda i: (0, 0)),
                  pl.BlockSpec((DIM, DIM), lambda i: (0, 0))],
        out_specs=[_bs_half(), _bs_half()],
        out_shape=[jax.ShapeDtypeStruct((P_NODES, HALF), _f32)] * 2,
    )(alo, ahi, glo, ghi, dinv, b1, W2)


def _final(alo, ahi, glo, ghi, dinv, b2):
    return pl.pallas_call(
        _tc_final,
        grid=(_GRID,),
        in_specs=[_bs_part(), _bs_part(), _bs_half(), _bs_half(), _bs_half(),
                  pl.BlockSpec((1, DIM), lambda i: (0, 0))],
        out_specs=pl.BlockSpec((_BLK, DIM), lambda i: (i, 0)),
        out_shape=jax.ShapeDtypeStruct((P_NODES, DIM), _f32),
    )(alo, ahi, glo, ghi, dinv, b2)


def kernel(edge_index, edge_pairs, emb, W1, b1, W2, b2):
    pad = jnp.full((EP - N_EDGES,), N_NODES, _i32)
    srcf = jnp.concatenate([edge_index[0], pad])
    dst2d = jnp.concatenate([edge_index[1], pad]).reshape(EROWS, 128)
    u2d = edge_pairs[0].reshape(PROWS, 128)
    v2d = edge_pairs[1].reshape(PROWS, 128)
    emb_p = jnp.pad(emb, ((0, P_NODES - N_NODES), (0, 0)))
    b1r = b1.reshape(1, DIM)
    b2r = b2.reshape(1, DIM)

    degp = _hist_kernel(dst2d)
    dinv, g1lo, g1hi = _prep1(degp, emb_p, W1)
    a1lo, a1hi = _scatter_kernel(srcf, dst2d, g1lo, g1hi)
    g2lo, g2hi = _mid(a1lo, a1hi, g1lo, g1hi, dinv, b1r, W2)
    a2lo, a2hi = _scatter_kernel(srcf, dst2d, g2lo, g2hi)
    z = _final(a2lo, a2hi, g2lo, g2hi, dinv, b2r)
    return _decode_kernel(z, u2d, v2d)


# speedup vs baseline: 17.6196x; 1.0749x over previous
import functools

import jax
import jax.numpy as jnp
from jax import lax
from jax.experimental import pallas as pl
from jax.experimental.pallas import tpu as pltpu
from jax.experimental.pallas import tpu_sc as plsc

N_NODES = 100000
P_NODES = 100352
DIM = 32
HALF = 16
N_EDGES = 1600000
EP = 1638400
EROWS = EP // 128
NW = 32
RW = EROWS // NW
SB = 16
NSB = RW // SB
CH = 512
EW0 = 73728
EW1 = 28672
E0 = EW0 * 16
NSB0 = EW0 // (4 * CH)
NSB1 = EW1 // (4 * CH)
RW0 = EW0 // 128
RW1 = EW1 // 128
ZROWS = P_NODES // 16
N_PAIRS = 262144
PROWS = N_PAIRS // 128
PR_W = PROWS // NW

_f32 = jnp.float32
_i32 = jnp.int32

_MESH = plsc.VectorSubcoreMesh(
    core_axis_name="c", subcore_axis_name="s", num_cores=2, num_subcores=16)


def _fill_rows(ref, nrows, value):
    def body(i, _):
        ref[i] = jnp.full((16,), value, _f32)
        return 0
    lax.fori_loop(0, nrows, body, 0)


@functools.partial(
    pl.kernel,
    out_type=jax.ShapeDtypeStruct((2, P_NODES, HALF), _f32),
    mesh=_MESH,
    compiler_params=pltpu.CompilerParams(use_tc_tiling_on_sc=False, needs_layout_passes=False),
    scratch_types=[
        pltpu.VMEM((SB, 128), _i32),
        pltpu.VMEM((SB, 128), _i32),
        pltpu.VMEM((128, HALF), _f32),
        pltpu.VMEM((128, HALF), _f32),
        pltpu.VMEM_SHARED((P_NODES, HALF), _f32),
        pltpu.SemaphoreType.DMA((2,)),
        pltpu.SemaphoreType.DMA,
    ],
)
def _hist_kernel(dst_hbm, out_hbm, dst0_v, dst1_v, ones_v, zero_v, acc_sh,
                 ssems, zsem):
    c = lax.axis_index("c")
    s = lax.axis_index("s")
    row0 = jnp.where(c == 0, s * RW0, E0 // 128 + s * RW1)
    nsb = jnp.where(c == 0, RW0 // SB, RW1 // SB)
    _fill_rows(ones_v, 128, 1.0)
    _fill_rows(zero_v, 128, 0.0)
    dbufs = (dst0_v, dst1_v)

    def zfire(i, _):
        pltpu.make_async_copy(
            zero_v, acc_sh.at[pl.ds(s * ZROWS + i * 128, 128)], zsem).start()
        return 0
    lax.fori_loop(0, ZROWS // 128, zfire, 0)

    def zdrain(i, _):
        pltpu.make_async_copy(
            zero_v, acc_sh.at[pl.ds(s * ZROWS, 128)], zsem).wait()
        return 0
    lax.fori_loop(0, ZROWS // 128, zdrain, 0)
    plsc.subcore_barrier()

    def stage(sb, buf):
        pltpu.sync_copy(dst_hbm.at[pl.ds(row0 + sb * SB, SB)], buf)

    def fire(buf, sem):
        for j in range(SB):
            pltpu.async_copy(ones_v, acc_sh.at[buf.at[j]], sem, add=True)

    def drain(buf, sem):
        for j in range(SB):
            pltpu.make_async_copy(ones_v, acc_sh.at[buf.at[j]], sem).wait()

    stage(0, dst0_v)
    fire(dst0_v, ssems.at[0])

    def sblock(p, _):
        sb = p * 2

        @pl.when(sb + 1 < nsb)
        def _():
            stage(sb + 1, dst1_v)
            fire(dst1_v, ssems.at[1])
        drain(dst0_v, ssems.at[0])

        @pl.when(sb + 2 < nsb)
        def _():
            stage(sb + 2, dst0_v)
            fire(dst0_v, ssems.at[0])

        @pl.when(sb + 1 < nsb)
        def _():
            drain(dst1_v, ssems.at[1])
        return 0
    lax.fori_loop(0, (nsb + 1) // 2, sblock, 0)
    plsc.subcore_barrier()
    pltpu.sync_copy(acc_sh.at[pl.ds(s * ZROWS, ZROWS)],
                    out_hbm.at[c, pl.ds(s * ZROWS, ZROWS)])


@functools.partial(
    pl.kernel,
    out_type=(jax.ShapeDtypeStruct((2, P_NODES, HALF), _f32),
              jax.ShapeDtypeStruct((2, P_NODES, HALF), _f32)),
    mesh=_MESH,
    compiler_params=pltpu.CompilerParams(use_tc_tiling_on_sc=False, needs_layout_passes=False),
    scratch_types=[
        pltpu.VMEM((4 * CH,), _i32),
        pltpu.VMEM((SB, 128), _i32),
        pltpu.VMEM((CH, HALF), _f32),
        pltpu.VMEM((CH, HALF), _f32),
        pltpu.VMEM((128, HALF), _f32),
        pltpu.VMEM_SHARED((P_NODES, HALF), _f32),
        pltpu.SemaphoreType.DMA((2,)),
        pltpu.SemaphoreType.DMA((2,)),
        pltpu.SemaphoreType.DMA,
    ],
)
def _scatter_kernel(srcf_hbm, dst_hbm, glo_hbm, ghi_hbm, outlo_hbm, outhi_hbm,
                    src_v, dst_v, rb0_v, rb1_v, zero_v, acc_sh,
                    gsems, ssems, zsem):
    c = lax.axis_index("c")
    s = lax.axis_index("s")
    e0 = jnp.where(c == 0, s * EW0, E0 + s * EW1)
    nsb = jnp.where(c == 0, NSB0, NSB1)
    _fill_rows(zero_v, 128, 0.0)
    rbufs = (rb0_v, rb1_v)

    def run_half(g_hbm, out_hbm):
        def zfire(i, _):
            pltpu.make_async_copy(
                zero_v, acc_sh.at[pl.ds(s * ZROWS + i * 128, 128)],
                zsem).start()
            return 0
        lax.fori_loop(0, ZROWS // 128, zfire, 0)

        def zdrain(i, _):
            pltpu.make_async_copy(
                zero_v, acc_sh.at[pl.ds(s * ZROWS, 128)], zsem).wait()
            return 0
        lax.fori_loop(0, ZROWS // 128, zdrain, 0)
        plsc.subcore_barrier()

        def gather(k, buf, sem):
            return pltpu.make_async_copy(
                g_hbm.at[src_v.at[pl.ds(k * CH, CH)]], buf, sem)

        def scat_start(k, q, sem):
            buf = rbufs[k % 2]
            pltpu.async_copy(buf.at[pl.ds(q * 128, 128)],
                             acc_sh.at[dst_v.at[k * 4 + q]], sem, add=True)

        def scat_wait(k, q, sem):
            buf = rbufs[k % 2]
            pltpu.make_async_copy(buf.at[pl.ds(q * 128, 128)],
                                  acc_sh.at[dst_v.at[k * 4 + q]], sem).wait()

        def sblock(sb, _):
            pltpu.sync_copy(srcf_hbm.at[pl.ds(e0 + sb * 4 * CH, 4 * CH)],
                            src_v)
            pltpu.sync_copy(dst_hbm.at[pl.ds(e0 // 128 + sb * SB, SB)], dst_v)
            gather(0, rb0_v, gsems.at[0]).start()
            gather(1, rb1_v, gsems.at[1]).start()
            for k in range(4):
                sl = k % 2
                gather(k, rbufs[sl], gsems.at[sl]).wait()
                for q in range(4):
                    scat_start(k, q, ssems.at[sl])
                if k + 2 < 4:
                    for q in range(4):
                        scat_wait(k, q, ssems.at[sl])
                    gather(k + 2, rbufs[sl], gsems.at[sl]).start()
            for k in (2, 3):
                for q in range(4):
                    scat_wait(k, q, ssems.at[k % 2])
            return 0
        lax.fori_loop(0, nsb, sblock, 0)
        plsc.subcore_barrier()
        pltpu.sync_copy(acc_sh.at[pl.ds(s * ZROWS, ZROWS)],
                        out_hbm.at[c, pl.ds(s * ZROWS, ZROWS)])
        plsc.subcore_barrier()

    run_half(glo_hbm, outlo_hbm)
    run_half(ghi_hbm, outhi_hbm)


@functools.partial(
    pl.kernel,
    out_type=jax.ShapeDtypeStruct((N_PAIRS,), _f32),
    mesh=_MESH,
    compiler_params=pltpu.CompilerParams(use_tc_tiling_on_sc=False, needs_layout_passes=False),
    scratch_types=[
        pltpu.VMEM((PR_W, 128), _i32),
        pltpu.VMEM((PR_W, 128), _i32),
        pltpu.VMEM((128, DIM), _f32),
        pltpu.VMEM((128, DIM), _f32),
        pltpu.VMEM((128, DIM), _f32),
        pltpu.VMEM((128, DIM), _f32),
        pltpu.VMEM((PR_W * 128,), _f32),
        pltpu.SemaphoreType.DMA((2,)),
        pltpu.SemaphoreType.DMA((2,)),
    ],
)
def _decode_kernel(z_hbm, u_hbm, v_hbm, scores_hbm,
                   u_v, v_v, zu0_v, zv0_v, zu1_v, zv1_v, sc_v, usems, vsems):
    c = lax.axis_index("c")
    s = lax.axis_index("s")
    w = c * 16 + s
    pltpu.sync_copy(u_hbm.at[pl.ds(w * PR_W, PR_W)], u_v)
    pltpu.sync_copy(v_hbm.at[pl.ds(w * PR_W, PR_W)], v_v)
    iota16 = lax.iota(_i32, 16)
    zubufs = (zu0_v, zu1_v)
    zvbufs = (zv0_v, zv1_v)

    def fetch(j, sl):
        return (pltpu.make_async_copy(z_hbm.at[u_v.at[j]], zubufs[sl],
                                      usems.at[sl]),
                pltpu.make_async_copy(z_hbm.at[v_v.at[j]], zvbufs[sl],
                                      vsems.at[sl]))

    def compute(j, sl):
        zu, zv = zubufs[sl], zvbufs[sl]

        def grp(g, _):
            ridx = g * 16 + iota16

            def feat(jf, acc):
                cidx = jnp.full((16,), jf, _i32)
                return acc + (plsc.load_gather(zu, [ridx, cidx]) *
                              plsc.load_gather(zv, [ridx, cidx]))
            sc = lax.fori_loop(0, DIM, feat, jnp.zeros((16,), _f32))
            sc_v[pl.ds(j * 128 + g * 16, 16)] = sc
            return 0
        lax.fori_loop(0, 8, grp, 0)

    for desc in fetch(0, 0):
        desc.start()
    for desc in fetch(1, 1):
        desc.start()

    def chunk(jp, _):
        j = jp * 2
        for desc in fetch(j, 0):
            desc.wait()
        compute(j, 0)

        @pl.when(j + 2 < PR_W)
        def _():
            for desc in fetch(j + 2, 0):
                desc.start()

        for desc in fetch(j + 1, 1):
            desc.wait()
        compute(j + 1, 1)

        @pl.when(j + 3 < PR_W)
        def _():
            for desc in fetch(j + 3, 1):
                desc.start()
        return 0
    lax.fori_loop(0, PR_W // 2, chunk, 0)
    pltpu.sync_copy(sc_v, scores_hbm.at[pl.ds(w * PR_W * 128, PR_W * 128)])


_BLK = 2048
_GRID = P_NODES // _BLK


def _row_mask(i, x):
    rows = lax.broadcasted_iota(_i32, (x.shape[0], 1), 0) + i * _BLK
    return jnp.where(rows < N_NODES, x, 0.0)


def _tc_prep1(degp_ref, emb_ref, w1_ref, dinv_ref, glo_ref, ghi_ref):
    i = pl.program_id(0)
    deg = degp_ref[0] + degp_ref[1] + 1.0
    dinv = lax.rsqrt(deg)
    dinv_ref[...] = dinv
    h = jnp.dot(emb_ref[...], w1_ref[...], preferred_element_type=_f32)
    g = _row_mask(i, h * dinv[:, :1])
    glo_ref[...] = g[:, :HALF]
    ghi_ref[...] = g[:, HALF:]


def _tc_mid(alo_ref, ahi_ref, glo_ref, ghi_ref, dinv_ref, b1_ref, w2_ref,
            g2lo_ref, g2hi_ref):
    i = pl.program_id(0)
    dinv = dinv_ref[...]
    olo = dinv * (alo_ref[0] + alo_ref[1] + glo_ref[...]) + b1_ref[0, :HALF]
    ohi = dinv * (ahi_ref[0] + ahi_ref[1] + ghi_ref[...]) + b1_ref[0, HALF:]
    z1 = jax.nn.relu(jnp.concatenate([olo, ohi], axis=1))
    h = jnp.dot(z1, w2_ref[...], preferred_element_type=_f32)
    g = _row_mask(i, h * dinv[:, :1])
    g2lo_ref[...] = g[:, :HALF]
    g2hi_ref[...] = g[:, HALF:]


def _tc_final(alo_ref, ahi_ref, glo_ref, ghi_ref, dinv_ref, b2_ref, z_ref):
    dinv = dinv_ref[...]
    zlo = dinv * (alo_ref[0] + alo_ref[1] + glo_ref[...]) + b2_ref[0, :HALF]
    zhi = dinv * (ahi_ref[0] + ahi_ref[1] + ghi_ref[...]) + b2_ref[0, HALF:]
    z_ref[...] = jnp.concatenate([zlo, zhi], axis=1)


def _bs_half():
    return pl.BlockSpec((_BLK, HALF), lambda i: (i, 0))


def _bs_part():
    return pl.BlockSpec((2, _BLK, HALF), lambda i: (0, i, 0))


def _prep1(degp, emb_p, W1):
    return pl.pallas_call(
        _tc_prep1,
        grid=(_GRID,),
        in_specs=[_bs_part(),
                  pl.BlockSpec((_BLK, DIM), lambda i: (i, 0)),
                  pl.BlockSpec((DIM, DIM), lambda i: (0, 0))],
        out_specs=[_bs_half(), _bs_half(), _bs_half()],
        out_shape=[jax.ShapeDtypeStruct((P_NODES, HALF), _f32)] * 3,
    )(degp, emb_p, W1)


def _mid(alo, ahi, glo, ghi, dinv, b1, W2):
    return pl.pallas_call(
        _tc_mid,
        grid=(_GRID,),
        in_specs=[_bs_part(), _bs_part(), _bs_half(), _bs_half(), _bs_half(),
                  pl.BlockSpec((1, DIM), lambda i: (0, 0)),
                  pl.BlockSpec((DIM, DIM), lambda i: (0, 0))],
        out_specs=[_bs_half(), _bs_half()],
        out_shape=[jax.ShapeDtypeStruct((P_NODES, HALF), _f32)] * 2,
    )(alo, ahi, glo, ghi, dinv, b1, W2)


def _final(alo, ahi, glo, ghi, dinv, b2):
    return pl.pallas_call(
        _tc_final,
        grid=(_GRID,),
        in_specs=[_bs_part(), _bs_part(), _bs_half(), _bs_half(), _bs_half(),
                  pl.BlockSpec((1, DIM), lambda i: (0, 0))],
        out_specs=pl.BlockSpec((_BLK, DIM), lambda i: (i, 0)),
        out_shape=jax.ShapeDtypeStruct((P_NODES, DIM), _f32),
    )(alo, ahi, glo, ghi, dinv, b2)


def kernel(edge_index, edge_pairs, emb, W1, b1, W2, b2):
    pad = jnp.full((EP - N_EDGES,), N_NODES, _i32)
    srcf = jnp.concatenate([edge_index[0], pad])
    dst2d = jnp.concatenate([edge_index[1], pad]).reshape(EROWS, 128)
    u2d = edge_pairs[0].reshape(PROWS, 128)
    v2d = edge_pairs[1].reshape(PROWS, 128)
    b1r = b1.reshape(1, DIM)
    b2r = b2.reshape(1, DIM)

    degp = _hist_kernel(dst2d)
    dinv, g1lo, g1hi = _prep1(degp, emb, W1)
    a1lo, a1hi = _scatter_kernel(srcf, dst2d, g1lo, g1hi)
    g2lo, g2hi = _mid(a1lo, a1hi, g1lo, g1hi, dinv, b1r, W2)
    a2lo, a2hi = _scatter_kernel(srcf, dst2d, g2lo, g2hi)
    z = _final(a2lo, a2hi, g2lo, g2hi, dinv, b2r)
    return _decode_kernel(z, u2d, v2d)


# speedup vs baseline: 22.0480x; 1.2513x over previous
import functools

import jax
import jax.numpy as jnp
from jax import lax
from jax.experimental import pallas as pl
from jax.experimental.pallas import tpu as pltpu
from jax.experimental.pallas import tpu_sc as plsc

N_NODES = 100000
P_NODES = 100352
DIM = 32
HALF = 16
N_EDGES = 1600000
EP = 1638400
EROWS = EP // 128
NW = 32
RW = EROWS // NW
SB = 16
NSB = RW // SB
CH = 512
EW0 = 73728
EW1 = 28672
E0 = EW0 * 16
NSB0 = EW0 // (4 * CH)
NSB1 = EW1 // (4 * CH)
RW0 = EW0 // 128
RW1 = EW1 // 128
ZROWS = P_NODES // 16
N_PAIRS = 262144
PROWS = N_PAIRS // 128
PR_W = PROWS // NW

_f32 = jnp.float32
_i32 = jnp.int32

_MESH = plsc.VectorSubcoreMesh(
    core_axis_name="c", subcore_axis_name="s", num_cores=2, num_subcores=16)


def _fill_rows(ref, nrows, value):
    def body(i, _):
        ref[i] = jnp.full((16,), value, _f32)
        return 0
    lax.fori_loop(0, nrows, body, 0)


@functools.partial(
    pl.kernel,
    out_type=jax.ShapeDtypeStruct((2, P_NODES, HALF), _f32),
    mesh=_MESH,
    compiler_params=pltpu.CompilerParams(use_tc_tiling_on_sc=False, needs_layout_passes=False),
    scratch_types=[
        pltpu.VMEM((SB, 128), _i32),
        pltpu.VMEM((SB, 128), _i32),
        pltpu.VMEM((128, HALF), _f32),
        pltpu.VMEM((128, HALF), _f32),
        pltpu.VMEM_SHARED((P_NODES, HALF), _f32),
        pltpu.SemaphoreType.DMA((2,)),
        pltpu.SemaphoreType.DMA,
    ],
)
def _hist_kernel(dst_hbm, out_hbm, dst0_v, dst1_v, ones_v, zero_v, acc_sh,
                 ssems, zsem):
    c = lax.axis_index("c")
    s = lax.axis_index("s")
    row0 = jnp.where(c == 0, s * RW0, E0 // 128 + s * RW1)
    nsb = jnp.where(c == 0, RW0 // SB, RW1 // SB)
    _fill_rows(ones_v, 128, 1.0)
    _fill_rows(zero_v, 128, 0.0)
    dbufs = (dst0_v, dst1_v)

    def zfire(i, _):
        pltpu.make_async_copy(
            zero_v, acc_sh.at[pl.ds(s * ZROWS + i * 128, 128)], zsem).start()
        return 0
    lax.fori_loop(0, ZROWS // 128, zfire, 0)

    def zdrain(i, _):
        pltpu.make_async_copy(
            zero_v, acc_sh.at[pl.ds(s * ZROWS, 128)], zsem).wait()
        return 0
    lax.fori_loop(0, ZROWS // 128, zdrain, 0)
    plsc.subcore_barrier()

    def stage(sb, buf):
        pltpu.sync_copy(dst_hbm.at[pl.ds(row0 + sb * SB, SB)], buf)

    def fire(buf, sem):
        for j in range(SB):
            pltpu.async_copy(ones_v, acc_sh.at[buf.at[j]], sem, add=True)

    def drain(buf, sem):
        for j in range(SB):
            pltpu.make_async_copy(ones_v, acc_sh.at[buf.at[j]], sem).wait()

    stage(0, dst0_v)
    fire(dst0_v, ssems.at[0])

    def sblock(p, _):
        sb = p * 2

        @pl.when(sb + 1 < nsb)
        def _():
            stage(sb + 1, dst1_v)
            fire(dst1_v, ssems.at[1])
        drain(dst0_v, ssems.at[0])

        @pl.when(sb + 2 < nsb)
        def _():
            stage(sb + 2, dst0_v)
            fire(dst0_v, ssems.at[0])

        @pl.when(sb + 1 < nsb)
        def _():
            drain(dst1_v, ssems.at[1])
        return 0
    lax.fori_loop(0, (nsb + 1) // 2, sblock, 0)
    plsc.subcore_barrier()
    pltpu.sync_copy(acc_sh.at[pl.ds(s * ZROWS, ZROWS)],
                    out_hbm.at[c, pl.ds(s * ZROWS, ZROWS)])


@functools.partial(
    pl.kernel,
    out_type=(jax.ShapeDtypeStruct((2, P_NODES, HALF), _f32),
              jax.ShapeDtypeStruct((2, P_NODES, HALF), _f32)),
    mesh=_MESH,
    compiler_params=pltpu.CompilerParams(use_tc_tiling_on_sc=False, needs_layout_passes=False),
    scratch_types=[
        pltpu.VMEM((4 * CH,), _i32),
        pltpu.VMEM((SB, 128), _i32),
        pltpu.VMEM((CH, HALF), _f32),
        pltpu.VMEM((CH, HALF), _f32),
        pltpu.VMEM((128, HALF), _f32),
        pltpu.VMEM_SHARED((P_NODES, HALF), _f32),
        pltpu.SemaphoreType.DMA((2,)),
        pltpu.SemaphoreType.DMA((2,)),
        pltpu.SemaphoreType.DMA,
    ],
)
def _scatter_kernel(srcf_hbm, dst_hbm, glo_hbm, ghi_hbm, outlo_hbm, outhi_hbm,
                    src_v, dst_v, rb0_v, rb1_v, zero_v, acc_sh,
                    gsems, ssems, zsem):
    c = lax.axis_index("c")
    s = lax.axis_index("s")
    e0 = jnp.where(c == 0, s * EW0, E0 + s * EW1)
    nsb = jnp.where(c == 0, NSB0, NSB1)
    _fill_rows(zero_v, 128, 0.0)
    rbufs = (rb0_v, rb1_v)

    def run_half(g_hbm, out_hbm):
        def zfire(i, _):
            pltpu.make_async_copy(
                zero_v, acc_sh.at[pl.ds(s * ZROWS + i * 128, 128)],
                zsem).start()
            return 0
        lax.fori_loop(0, ZROWS // 128, zfire, 0)

        def zdrain(i, _):
            pltpu.make_async_copy(
                zero_v, acc_sh.at[pl.ds(s * ZROWS, 128)], zsem).wait()
            return 0
        lax.fori_loop(0, ZROWS // 128, zdrain, 0)
        plsc.subcore_barrier()

        def gather(k, buf, sem):
            return pltpu.make_async_copy(
                g_hbm.at[src_v.at[pl.ds(k * CH, CH)]], buf, sem)

        def scat_start(k, q, sem):
            buf = rbufs[k % 2]
            pltpu.async_copy(buf.at[pl.ds(q * 128, 128)],
                             acc_sh.at[dst_v.at[k * 4 + q]], sem, add=True)

        def scat_wait(k, q, sem):
            buf = rbufs[k % 2]
            pltpu.make_async_copy(buf.at[pl.ds(q * 128, 128)],
                                  acc_sh.at[dst_v.at[k * 4 + q]], sem).wait()

        def sblock(sb, _):
            pltpu.sync_copy(srcf_hbm.at[pl.ds(e0 + sb * 4 * CH, 4 * CH)],
                            src_v)
            pltpu.sync_copy(dst_hbm.at[pl.ds(e0 // 128 + sb * SB, SB)], dst_v)
            gather(0, rb0_v, gsems.at[0]).start()
            gather(1, rb1_v, gsems.at[1]).start()
            for k in range(4):
                sl = k % 2
                gather(k, rbufs[sl], gsems.at[sl]).wait()
                for q in range(4):
                    scat_start(k, q, ssems.at[sl])
                if k + 2 < 4:
                    for q in range(4):
                        scat_wait(k, q, ssems.at[sl])
                    gather(k + 2, rbufs[sl], gsems.at[sl]).start()
            for k in (2, 3):
                for q in range(4):
                    scat_wait(k, q, ssems.at[k % 2])
            return 0
        lax.fori_loop(0, nsb, sblock, 0)
        plsc.subcore_barrier()
        pltpu.sync_copy(acc_sh.at[pl.ds(s * ZROWS, ZROWS)],
                        out_hbm.at[c, pl.ds(s * ZROWS, ZROWS)])
        plsc.subcore_barrier()

    run_half(glo_hbm, outlo_hbm)
    run_half(ghi_hbm, outhi_hbm)


@functools.partial(
    pl.kernel,
    out_type=jax.ShapeDtypeStruct((N_PAIRS,), _f32),
    mesh=_MESH,
    compiler_params=pltpu.CompilerParams(use_tc_tiling_on_sc=False, needs_layout_passes=False),
    scratch_types=[
        pltpu.VMEM((PR_W, 128), _i32),
        pltpu.VMEM((PR_W, 128), _i32),
        pltpu.VMEM((128, DIM), _f32),
        pltpu.VMEM((128, DIM), _f32),
        pltpu.VMEM((128, DIM), _f32),
        pltpu.VMEM((128, DIM), _f32),
        pltpu.VMEM((PR_W * 128,), _f32),
        pltpu.SemaphoreType.DMA((2,)),
        pltpu.SemaphoreType.DMA((2,)),
    ],
)
def _decode_kernel(z_hbm, u_hbm, v_hbm, scores_hbm,
                   u_v, v_v, zu0_v, zv0_v, zu1_v, zv1_v, sc_v, usems, vsems):
    c = lax.axis_index("c")
    s = lax.axis_index("s")
    w = c * 16 + s
    pltpu.sync_copy(u_hbm.at[pl.ds(w * PR_W, PR_W)], u_v)
    pltpu.sync_copy(v_hbm.at[pl.ds(w * PR_W, PR_W)], v_v)
    iota16 = lax.iota(_i32, 16)
    zubufs = (zu0_v, zu1_v)
    zvbufs = (zv0_v, zv1_v)

    def fetch(j, sl):
        return (pltpu.make_async_copy(z_hbm.at[u_v.at[j]], zubufs[sl],
                                      usems.at[sl]),
                pltpu.make_async_copy(z_hbm.at[v_v.at[j]], zvbufs[sl],
                                      vsems.at[sl]))

    def compute(j, sl):
        zu, zv = zubufs[sl], zvbufs[sl]

        def grp(g, _):
            ridx = g * 16 + iota16

            def feat(jf, acc):
                cidx = jnp.full((16,), jf, _i32)
                return acc + (plsc.load_gather(zu, [ridx, cidx]) *
                              plsc.load_gather(zv, [ridx, cidx]))
            sc = lax.fori_loop(0, DIM, feat, jnp.zeros((16,), _f32))
            sc_v[pl.ds(j * 128 + g * 16, 16)] = sc
            return 0
        lax.fori_loop(0, 8, grp, 0)

    for desc in fetch(0, 0):
        desc.start()
    for desc in fetch(1, 1):
        desc.start()

    def chunk(jp, _):
        j = jp * 2
        for desc in fetch(j, 0):
            desc.wait()
        compute(j, 0)

        @pl.when(j + 2 < PR_W)
        def _():
            for desc in fetch(j + 2, 0):
                desc.start()

        for desc in fetch(j + 1, 1):
            desc.wait()
        compute(j + 1, 1)

        @pl.when(j + 3 < PR_W)
        def _():
            for desc in fetch(j + 3, 1):
                desc.start()
        return 0
    lax.fori_loop(0, PR_W // 2, chunk, 0)
    pltpu.sync_copy(sc_v, scores_hbm.at[pl.ds(w * PR_W * 128, PR_W * 128)])


_BLK = 2048
_GRID = P_NODES // _BLK


def _row_mask(i, x):
    rows = lax.broadcasted_iota(_i32, (x.shape[0], 1), 0) + i * _BLK
    return jnp.where(rows < N_NODES, x, 0.0)


def _tc_prep1(degp_ref, emb_ref, w1_ref, dinv_ref, glo_ref, ghi_ref):
    i = pl.program_id(0)
    deg = degp_ref[0] + degp_ref[1] + 1.0
    dinv = lax.rsqrt(deg)
    dinv_ref[...] = dinv
    h = jnp.dot(emb_ref[...], w1_ref[...], preferred_element_type=_f32)
    g = _row_mask(i, h * dinv[:, :1])
    glo_ref[...] = g[:, :HALF]
    ghi_ref[...] = g[:, HALF:]


def _tc_mid(alo_ref, ahi_ref, glo_ref, ghi_ref, dinv_ref, b1_ref, w2_ref,
            g2lo_ref, g2hi_ref):
    i = pl.program_id(0)
    dinv = dinv_ref[...]
    olo = dinv * (alo_ref[0] + alo_ref[1] + glo_ref[...]) + b1_ref[0, :HALF]
    ohi = dinv * (ahi_ref[0] + ahi_ref[1] + ghi_ref[...]) + b1_ref[0, HALF:]
    z1 = jax.nn.relu(jnp.concatenate([olo, ohi], axis=1))
    h = jnp.dot(z1, w2_ref[...], preferred_element_type=_f32)
    g = _row_mask(i, h * dinv[:, :1])
    g2lo_ref[...] = g[:, :HALF]
    g2hi_ref[...] = g[:, HALF:]


def _tc_final(alo_ref, ahi_ref, glo_ref, ghi_ref, dinv_ref, b2_ref, z_ref):
    dinv = dinv_ref[...]
    zlo = dinv * (alo_ref[0] + alo_ref[1] + glo_ref[...]) + b2_ref[0, :HALF]
    zhi = dinv * (ahi_ref[0] + ahi_ref[1] + ghi_ref[...]) + b2_ref[0, HALF:]
    z_ref[...] = jnp.concatenate([zlo, zhi], axis=1)


def _bs_half():
    return pl.BlockSpec((_BLK, HALF), lambda i: (i, 0))


def _bs_part():
    return pl.BlockSpec((2, _BLK, HALF), lambda i: (0, i, 0))


def _prep1(degp, emb_p, W1):
    return pl.pallas_call(
        _tc_prep1,
        grid=(_GRID,),
        in_specs=[_bs_part(),
                  pl.BlockSpec((_BLK, DIM), lambda i: (i, 0)),
                  pl.BlockSpec((DIM, DIM), lambda i: (0, 0))],
        out_specs=[_bs_half(), _bs_half(), _bs_half()],
        out_shape=[jax.ShapeDtypeStruct((P_NODES, HALF), _f32)] * 3,
    )(degp, emb_p, W1)


def _mid(alo, ahi, glo, ghi, dinv, b1, W2):
    return pl.pallas_call(
        _tc_mid,
        grid=(_GRID,),
        in_specs=[_bs_part(), _bs_part(), _bs_half(), _bs_half(), _bs_half(),
                  pl.BlockSpec((1, DIM), lambda i: (0, 0)),
                  pl.BlockSpec((DIM, DIM), lambda i: (0, 0))],
        out_specs=[_bs_half(), _bs_half()],
        out_shape=[jax.ShapeDtypeStruct((P_NODES, HALF), _f32)] * 2,
    )(alo, ahi, glo, ghi, dinv, b1, W2)


def _final(alo, ahi, glo, ghi, dinv, b2):
    return pl.pallas_call(
        _tc_final,
        grid=(_GRID,),
        in_specs=[_bs_part(), _bs_part(), _bs_half(), _bs_half(), _bs_half(),
                  pl.BlockSpec((1, DIM), lambda i: (0, 0))],
        out_specs=pl.BlockSpec((_BLK, DIM), lambda i: (i, 0)),
        out_shape=jax.ShapeDtypeStruct((P_NODES, DIM), _f32),
    )(alo, ahi, glo, ghi, dinv, b2)


def kernel(edge_index, edge_pairs, emb, W1, b1, W2, b2):
    pad = N_NODES + jnp.arange(EP - N_EDGES, dtype=_i32) % (P_NODES - N_NODES)
    srcf = jnp.concatenate([edge_index[0], pad])
    dst2d = jnp.concatenate([edge_index[1], pad]).reshape(EROWS, 128)
    u2d = edge_pairs[0].reshape(PROWS, 128)
    v2d = edge_pairs[1].reshape(PROWS, 128)
    b1r = b1.reshape(1, DIM)
    b2r = b2.reshape(1, DIM)

    degp = _hist_kernel(dst2d)
    dinv, g1lo, g1hi = _prep1(degp, emb, W1)
    a1lo, a1hi = _scatter_kernel(srcf, dst2d, g1lo, g1hi)
    g2lo, g2hi = _mid(a1lo, a1hi, g1lo, g1hi, dinv, b1r, W2)
    a2lo, a2hi = _scatter_kernel(srcf, dst2d, g2lo, g2hi)
    z = _final(a2lo, a2hi, g2lo, g2hi, dinv, b2r)
    return _decode_kernel(z, u2d, v2d)


# speedup vs baseline: 24.8798x; 1.1284x over previous
import functools

import jax
import jax.numpy as jnp
from jax import lax
from jax.experimental import pallas as pl
from jax.experimental.pallas import tpu as pltpu
from jax.experimental.pallas import tpu_sc as plsc

N_NODES = 100000
P_NODES = 100352
DIM = 32
HALF = 16
N_EDGES = 1600000
EP = 1638400
EROWS = EP // 128
NW = 32
RW = EROWS // NW
SB = 16
NSB = RW // SB
CH = 512
EW0 = 51200
EW1 = 51200
E0 = EW0 * 16
NSB0 = EW0 // (4 * CH)
NSB1 = EW1 // (4 * CH)
RW0 = EW0 // 128
RW1 = EW1 // 128
ZROWS = P_NODES // 16
N_PAIRS = 262144
PROWS = N_PAIRS // 128
PR_W = PROWS // NW

_f32 = jnp.float32
_i32 = jnp.int32

_MESH = plsc.VectorSubcoreMesh(
    core_axis_name="c", subcore_axis_name="s", num_cores=2, num_subcores=16)


def _fill_rows(ref, nrows, value):
    def body(i, _):
        ref[i] = jnp.full((16,), value, _f32)
        return 0
    lax.fori_loop(0, nrows, body, 0)


@functools.partial(
    pl.kernel,
    out_type=jax.ShapeDtypeStruct((2, P_NODES, HALF), _f32),
    mesh=_MESH,
    compiler_params=pltpu.CompilerParams(use_tc_tiling_on_sc=False, needs_layout_passes=False),
    scratch_types=[
        pltpu.VMEM((SB, 128), _i32),
        pltpu.VMEM((SB, 128), _i32),
        pltpu.VMEM((128, HALF), _f32),
        pltpu.VMEM((128, HALF), _f32),
        pltpu.VMEM_SHARED((P_NODES, HALF), _f32),
        pltpu.SemaphoreType.DMA((2,)),
        pltpu.SemaphoreType.DMA,
    ],
)
def _hist_kernel(dst_hbm, out_hbm, dst0_v, dst1_v, ones_v, zero_v, acc_sh,
                 ssems, zsem):
    c = lax.axis_index("c")
    s = lax.axis_index("s")
    row0 = jnp.where(c == 0, s * RW0, E0 // 128 + s * RW1)
    nsb = jnp.where(c == 0, RW0 // SB, RW1 // SB)
    _fill_rows(ones_v, 128, 1.0)
    _fill_rows(zero_v, 128, 0.0)
    dbufs = (dst0_v, dst1_v)

    def zfire(i, _):
        pltpu.make_async_copy(
            zero_v, acc_sh.at[pl.ds(s * ZROWS + i * 128, 128)], zsem).start()
        return 0
    lax.fori_loop(0, ZROWS // 128, zfire, 0)

    def zdrain(i, _):
        pltpu.make_async_copy(
            zero_v, acc_sh.at[pl.ds(s * ZROWS, 128)], zsem).wait()
        return 0
    lax.fori_loop(0, ZROWS // 128, zdrain, 0)
    plsc.subcore_barrier()

    def stage(sb, buf):
        pltpu.sync_copy(dst_hbm.at[pl.ds(row0 + sb * SB, SB)], buf)

    def fire(buf, sem):
        for j in range(SB):
            pltpu.async_copy(ones_v, acc_sh.at[buf.at[j]], sem, add=True)

    def drain(buf, sem):
        for j in range(SB):
            pltpu.make_async_copy(ones_v, acc_sh.at[buf.at[j]], sem).wait()

    stage(0, dst0_v)
    fire(dst0_v, ssems.at[0])

    def sblock(p, _):
        sb = p * 2

        @pl.when(sb + 1 < nsb)
        def _():
            stage(sb + 1, dst1_v)
            fire(dst1_v, ssems.at[1])
        drain(dst0_v, ssems.at[0])

        @pl.when(sb + 2 < nsb)
        def _():
            stage(sb + 2, dst0_v)
            fire(dst0_v, ssems.at[0])

        @pl.when(sb + 1 < nsb)
        def _():
            drain(dst1_v, ssems.at[1])
        return 0
    lax.fori_loop(0, (nsb + 1) // 2, sblock, 0)
    plsc.subcore_barrier()
    pltpu.sync_copy(acc_sh.at[pl.ds(s * ZROWS, ZROWS)],
                    out_hbm.at[c, pl.ds(s * ZROWS, ZROWS)])


@functools.partial(
    pl.kernel,
    out_type=(jax.ShapeDtypeStruct((2, P_NODES, HALF), _f32),
              jax.ShapeDtypeStruct((2, P_NODES, HALF), _f32)),
    mesh=_MESH,
    compiler_params=pltpu.CompilerParams(use_tc_tiling_on_sc=False, needs_layout_passes=False),
    scratch_types=[
        pltpu.VMEM((4 * CH,), _i32),
        pltpu.VMEM((SB, 128), _i32),
        pltpu.VMEM((CH, HALF), _f32),
        pltpu.VMEM((CH, HALF), _f32),
        pltpu.VMEM((128, HALF), _f32),
        pltpu.VMEM_SHARED((P_NODES, HALF), _f32),
        pltpu.SemaphoreType.DMA((2,)),
        pltpu.SemaphoreType.DMA((2,)),
        pltpu.SemaphoreType.DMA,
    ],
)
def _scatter_kernel(srcf_hbm, dst_hbm, glo_hbm, ghi_hbm, outlo_hbm, outhi_hbm,
                    src_v, dst_v, rb0_v, rb1_v, zero_v, acc_sh,
                    gsems, ssems, zsem):
    c = lax.axis_index("c")
    s = lax.axis_index("s")
    e0 = jnp.where(c == 0, s * EW0, E0 + s * EW1)
    nsb = jnp.where(c == 0, NSB0, NSB1)
    _fill_rows(zero_v, 128, 0.0)
    rbufs = (rb0_v, rb1_v)

    def run_half(g_hbm, out_hbm):
        def zfire(i, _):
            pltpu.make_async_copy(
                zero_v, acc_sh.at[pl.ds(s * ZROWS + i * 128, 128)],
                zsem).start()
            return 0
        lax.fori_loop(0, ZROWS // 128, zfire, 0)

        def zdrain(i, _):
            pltpu.make_async_copy(
                zero_v, acc_sh.at[pl.ds(s * ZROWS, 128)], zsem).wait()
            return 0
        lax.fori_loop(0, ZROWS // 128, zdrain, 0)
        plsc.subcore_barrier()

        def gather(k, buf, sem):
            return pltpu.make_async_copy(
                g_hbm.at[src_v.at[pl.ds(k * CH, CH)]], buf, sem)

        def scat_start(k, q, sem):
            buf = rbufs[k % 2]
            pltpu.async_copy(buf.at[pl.ds(q * 128, 128)],
                             acc_sh.at[dst_v.at[k * 4 + q]], sem, add=True)

        def scat_wait(k, q, sem):
            buf = rbufs[k % 2]
            pltpu.make_async_copy(buf.at[pl.ds(q * 128, 128)],
                                  acc_sh.at[dst_v.at[k * 4 + q]], sem).wait()

        def sblock(sb, _):
            pltpu.sync_copy(srcf_hbm.at[pl.ds(e0 + sb * 4 * CH, 4 * CH)],
                            src_v)
            pltpu.sync_copy(dst_hbm.at[pl.ds(e0 // 128 + sb * SB, SB)], dst_v)
            gather(0, rb0_v, gsems.at[0]).start()
            gather(1, rb1_v, gsems.at[1]).start()
            for k in range(4):
                sl = k % 2
                gather(k, rbufs[sl], gsems.at[sl]).wait()
                for q in range(4):
                    scat_start(k, q, ssems.at[sl])
                if k + 2 < 4:
                    for q in range(4):
                        scat_wait(k, q, ssems.at[sl])
                    gather(k + 2, rbufs[sl], gsems.at[sl]).start()
            for k in (2, 3):
                for q in range(4):
                    scat_wait(k, q, ssems.at[k % 2])
            return 0
        lax.fori_loop(0, nsb, sblock, 0)
        plsc.subcore_barrier()
        pltpu.sync_copy(acc_sh.at[pl.ds(s * ZROWS, ZROWS)],
                        out_hbm.at[c, pl.ds(s * ZROWS, ZROWS)])
        plsc.subcore_barrier()

    run_half(glo_hbm, outlo_hbm)
    run_half(ghi_hbm, outhi_hbm)


@functools.partial(
    pl.kernel,
    out_type=jax.ShapeDtypeStruct((N_PAIRS,), _f32),
    mesh=_MESH,
    compiler_params=pltpu.CompilerParams(use_tc_tiling_on_sc=False, needs_layout_passes=False),
    scratch_types=[
        pltpu.VMEM((PR_W, 128), _i32),
        pltpu.VMEM((PR_W, 128), _i32),
        pltpu.VMEM((128, DIM), _f32),
        pltpu.VMEM((128, DIM), _f32),
        pltpu.VMEM((128, DIM), _f32),
        pltpu.VMEM((128, DIM), _f32),
        pltpu.VMEM((PR_W * 128,), _f32),
        pltpu.SemaphoreType.DMA((2,)),
        pltpu.SemaphoreType.DMA((2,)),
    ],
)
def _decode_kernel(z_hbm, u_hbm, v_hbm, scores_hbm,
                   u_v, v_v, zu0_v, zv0_v, zu1_v, zv1_v, sc_v, usems, vsems):
    c = lax.axis_index("c")
    s = lax.axis_index("s")
    w = c * 16 + s
    pltpu.sync_copy(u_hbm.at[pl.ds(w * PR_W, PR_W)], u_v)
    pltpu.sync_copy(v_hbm.at[pl.ds(w * PR_W, PR_W)], v_v)
    iota16 = lax.iota(_i32, 16)
    zubufs = (zu0_v, zu1_v)
    zvbufs = (zv0_v, zv1_v)

    def fetch(j, sl):
        return (pltpu.make_async_copy(z_hbm.at[u_v.at[j]], zubufs[sl],
                                      usems.at[sl]),
                pltpu.make_async_copy(z_hbm.at[v_v.at[j]], zvbufs[sl],
                                      vsems.at[sl]))

    def compute(j, sl):
        zu, zv = zubufs[sl], zvbufs[sl]

        def grp(g, _):
            ridx = g * 16 + iota16
            sc = jnp.zeros((16,), _f32)
            for jf in range(DIM):
                cidx = jnp.full((16,), jf, _i32)
                sc = sc + (plsc.load_gather(zu, [ridx, cidx]) *
                           plsc.load_gather(zv, [ridx, cidx]))
            sc_v[pl.ds(j * 128 + g * 16, 16)] = sc
            return 0
        lax.fori_loop(0, 8, grp, 0)

    for desc in fetch(0, 0):
        desc.start()
    for desc in fetch(1, 1):
        desc.start()

    def chunk(jp, _):
        j = jp * 2
        for desc in fetch(j, 0):
            desc.wait()
        compute(j, 0)

        @pl.when(j + 2 < PR_W)
        def _():
            for desc in fetch(j + 2, 0):
                desc.start()

        for desc in fetch(j + 1, 1):
            desc.wait()
        compute(j + 1, 1)

        @pl.when(j + 3 < PR_W)
        def _():
            for desc in fetch(j + 3, 1):
                desc.start()
        return 0
    lax.fori_loop(0, PR_W // 2, chunk, 0)
    pltpu.sync_copy(sc_v, scores_hbm.at[pl.ds(w * PR_W * 128, PR_W * 128)])


_BLK = 2048
_GRID = P_NODES // _BLK


def _row_mask(i, x):
    rows = lax.broadcasted_iota(_i32, (x.shape[0], 1), 0) + i * _BLK
    return jnp.where(rows < N_NODES, x, 0.0)


def _tc_prep1(degp_ref, emb_ref, w1_ref, dinv_ref, glo_ref, ghi_ref):
    i = pl.program_id(0)
    deg = degp_ref[0] + degp_ref[1] + 1.0
    dinv = lax.rsqrt(deg)
    dinv_ref[...] = dinv
    h = jnp.dot(emb_ref[...], w1_ref[...], preferred_element_type=_f32)
    g = _row_mask(i, h * dinv[:, :1])
    glo_ref[...] = g[:, :HALF]
    ghi_ref[...] = g[:, HALF:]


def _tc_mid(alo_ref, ahi_ref, glo_ref, ghi_ref, dinv_ref, b1_ref, w2_ref,
            g2lo_ref, g2hi_ref):
    i = pl.program_id(0)
    dinv = dinv_ref[...]
    olo = dinv * (alo_ref[0] + alo_ref[1] + glo_ref[...]) + b1_ref[0, :HALF]
    ohi = dinv * (ahi_ref[0] + ahi_ref[1] + ghi_ref[...]) + b1_ref[0, HALF:]
    z1 = jax.nn.relu(jnp.concatenate([olo, ohi], axis=1))
    h = jnp.dot(z1, w2_ref[...], preferred_element_type=_f32)
    g = _row_mask(i, h * dinv[:, :1])
    g2lo_ref[...] = g[:, :HALF]
    g2hi_ref[...] = g[:, HALF:]


def _tc_final(alo_ref, ahi_ref, glo_ref, ghi_ref, dinv_ref, b2_ref, z_ref):
    dinv = dinv_ref[...]
    zlo = dinv * (alo_ref[0] + alo_ref[1] + glo_ref[...]) + b2_ref[0, :HALF]
    zhi = dinv * (ahi_ref[0] + ahi_ref[1] + ghi_ref[...]) + b2_ref[0, HALF:]
    z_ref[...] = jnp.concatenate([zlo, zhi], axis=1)


def _bs_half():
    return pl.BlockSpec((_BLK, HALF), lambda i: (i, 0))


def _bs_part():
    return pl.BlockSpec((2, _BLK, HALF), lambda i: (0, i, 0))


def _prep1(degp, emb_p, W1):
    return pl.pallas_call(
        _tc_prep1,
        grid=(_GRID,),
        in_specs=[_bs_part(),
                  pl.BlockSpec((_BLK, DIM), lambda i: (i, 0)),
                  pl.BlockSpec((DIM, DIM), lambda i: (0, 0))],
        out_specs=[_bs_half(), _bs_half(), _bs_half()],
        out_shape=[jax.ShapeDtypeStruct((P_NODES, HALF), _f32)] * 3,
    )(degp, emb_p, W1)


def _mid(alo, ahi, glo, ghi, dinv, b1, W2):
    return pl.pallas_call(
        _tc_mid,
        grid=(_GRID,),
        in_specs=[_bs_part(), _bs_part(), _bs_half(), _bs_half(), _bs_half(),
                  pl.BlockSpec((1, DIM), lambda i: (0, 0)),
                  pl.BlockSpec((DIM, DIM), lambda i: (0, 0))],
        out_specs=[_bs_half(), _bs_half()],
        out_shape=[jax.ShapeDtypeStruct((P_NODES, HALF), _f32)] * 2,
    )(alo, ahi, glo, ghi, dinv, b1, W2)


def _final(alo, ahi, glo, ghi, dinv, b2):
    return pl.pallas_call(
        _tc_final,
        grid=(_GRID,),
        in_specs=[_bs_part(), _bs_part(), _bs_half(), _bs_half(), _bs_half(),
                  pl.BlockSpec((1, DIM), lambda i: (0, 0))],
        out_specs=pl.BlockSpec((_BLK, DIM), lambda i: (i, 0)),
        out_shape=jax.ShapeDtypeStruct((P_NODES, DIM), _f32),
    )(alo, ahi, glo, ghi, dinv, b2)


def kernel(edge_index, edge_pairs, emb, W1, b1, W2, b2):
    pad = N_NODES + jnp.arange(EP - N_EDGES, dtype=_i32) % (P_NODES - N_NODES)
    srcf = jnp.concatenate([edge_index[0], pad])
    dst2d = jnp.concatenate([edge_index[1], pad]).reshape(EROWS, 128)
    u2d = edge_pairs[0].reshape(PROWS, 128)
    v2d = edge_pairs[1].reshape(PROWS, 128)
    b1r = b1.reshape(1, DIM)
    b2r = b2.reshape(1, DIM)

    degp = _hist_kernel(dst2d)
    dinv, g1lo, g1hi = _prep1(degp, emb, W1)
    a1lo, a1hi = _scatter_kernel(srcf, dst2d, g1lo, g1hi)
    g2lo, g2hi = _mid(a1lo, a1hi, g1lo, g1hi, dinv, b1r, W2)
    a2lo, a2hi = _scatter_kernel(srcf, dst2d, g2lo, g2hi)
    z = _final(a2lo, a2hi, g2lo, g2hi, dinv, b2r)
    return _decode_kernel(z, u2d, v2d)


# speedup vs baseline: 46.9273x; 1.8862x over previous
import functools

import jax
import jax.numpy as jnp
from jax import lax
from jax.experimental import pallas as pl
from jax.experimental.pallas import tpu as pltpu
from jax.experimental.pallas import tpu_sc as plsc

N_NODES = 100000
P_NODES = 100352
DIM = 32
HALF = 16
N_EDGES = 1600000
EP = 1638400
EROWS = EP // 128
NW = 32
RW = EROWS // NW
SB = 16
NSB = RW // SB
CH = 512
EW0 = 51200
EW1 = 51200
E0 = EW0 * 16
NSB0 = EW0 // (4 * CH)
NSB1 = EW1 // (4 * CH)
RW0 = EW0 // 128
RW1 = EW1 // 128
ZROWS = P_NODES // 16
N_PAIRS = 262144
PROWS = N_PAIRS // 128
PR_W = PROWS // NW
NB = P_NODES * HALF // 128

_f32 = jnp.float32
_i32 = jnp.int32

_MESH = plsc.VectorSubcoreMesh(
    core_axis_name="c", subcore_axis_name="s", num_cores=2, num_subcores=16)


def _fill_rows(ref, nrows, value):
    def body(i, _):
        ref[i] = jnp.full((16,), value, _f32)
        return 0
    lax.fori_loop(0, nrows, body, 0)


@functools.partial(
    pl.kernel,
    out_type=jax.ShapeDtypeStruct((2, P_NODES, HALF), _f32),
    mesh=_MESH,
    compiler_params=pltpu.CompilerParams(use_tc_tiling_on_sc=False, needs_layout_passes=False),
    scratch_types=[
        pltpu.VMEM((SB, 128), _i32),
        pltpu.VMEM((SB, 128), _i32),
        pltpu.VMEM((128, HALF), _f32),
        pltpu.VMEM((128, HALF), _f32),
        pltpu.VMEM_SHARED((P_NODES, HALF), _f32),
        pltpu.SemaphoreType.DMA((2,)),
        pltpu.SemaphoreType.DMA,
    ],
)
def _hist_kernel(dst_hbm, out_hbm, dst0_v, dst1_v, ones_v, zero_v, acc_sh,
                 ssems, zsem):
    c = lax.axis_index("c")
    s = lax.axis_index("s")
    row0 = jnp.where(c == 0, s * RW0, E0 // 128 + s * RW1)
    nsb = jnp.where(c == 0, RW0 // SB, RW1 // SB)
    _fill_rows(ones_v, 128, 1.0)
    _fill_rows(zero_v, 128, 0.0)
    dbufs = (dst0_v, dst1_v)

    def zfire(i, _):
        pltpu.make_async_copy(
            zero_v, acc_sh.at[pl.ds(s * ZROWS + i * 128, 128)], zsem).start()
        return 0
    lax.fori_loop(0, ZROWS // 128, zfire, 0)

    def zdrain(i, _):
        pltpu.make_async_copy(
            zero_v, acc_sh.at[pl.ds(s * ZROWS, 128)], zsem).wait()
        return 0
    lax.fori_loop(0, ZROWS // 128, zdrain, 0)
    plsc.subcore_barrier()

    def stage(sb, buf):
        pltpu.sync_copy(dst_hbm.at[pl.ds(row0 + sb * SB, SB)], buf)

    def fire(buf, sem):
        for j in range(SB):
            pltpu.async_copy(ones_v, acc_sh.at[buf.at[j]], sem, add=True)

    def drain(buf, sem):
        for j in range(SB):
            pltpu.make_async_copy(ones_v, acc_sh.at[buf.at[j]], sem).wait()

    stage(0, dst0_v)
    fire(dst0_v, ssems.at[0])

    def sblock(p, _):
        sb = p * 2

        @pl.when(sb + 1 < nsb)
        def _():
            stage(sb + 1, dst1_v)
            fire(dst1_v, ssems.at[1])
        drain(dst0_v, ssems.at[0])

        @pl.when(sb + 2 < nsb)
        def _():
            stage(sb + 2, dst0_v)
            fire(dst0_v, ssems.at[0])

        @pl.when(sb + 1 < nsb)
        def _():
            drain(dst1_v, ssems.at[1])
        return 0
    lax.fori_loop(0, (nsb + 1) // 2, sblock, 0)
    plsc.subcore_barrier()
    pltpu.sync_copy(acc_sh.at[pl.ds(s * ZROWS, ZROWS)],
                    out_hbm.at[c, pl.ds(s * ZROWS, ZROWS)])


@functools.partial(
    pl.kernel,
    out_type=(jax.ShapeDtypeStruct((2, P_NODES, HALF), _f32),
              jax.ShapeDtypeStruct((2, P_NODES, HALF), _f32)),
    mesh=_MESH,
    compiler_params=pltpu.CompilerParams(use_tc_tiling_on_sc=False, needs_layout_passes=False),
    scratch_types=[
        pltpu.VMEM((4 * CH,), _i32),
        pltpu.VMEM((SB, 128), _i32),
        pltpu.VMEM((CH, HALF), _f32),
        pltpu.VMEM((CH, HALF), _f32),
        pltpu.VMEM((128, HALF), _f32),
        pltpu.VMEM_SHARED((P_NODES, HALF), _f32),
        pltpu.SemaphoreType.DMA((2,)),
        pltpu.SemaphoreType.DMA((2,)),
        pltpu.SemaphoreType.DMA,
    ],
)
def _scatter_kernel(srcf_hbm, dst_hbm, glo_hbm, ghi_hbm, outlo_hbm, outhi_hbm,
                    src_v, dst_v, rb0_v, rb1_v, zero_v, acc_sh,
                    gsems, ssems, zsem):
    c = lax.axis_index("c")
    s = lax.axis_index("s")
    e0 = jnp.where(c == 0, s * EW0, E0 + s * EW1)
    nsb = jnp.where(c == 0, NSB0, NSB1)
    _fill_rows(zero_v, 128, 0.0)
    rbufs = (rb0_v, rb1_v)

    def run_half(g_hbm, out_hbm):
        def zfire(i, _):
            pltpu.make_async_copy(
                zero_v, acc_sh.at[pl.ds(s * ZROWS + i * 128, 128)],
                zsem).start()
            return 0
        lax.fori_loop(0, ZROWS // 128, zfire, 0)

        def zdrain(i, _):
            pltpu.make_async_copy(
                zero_v, acc_sh.at[pl.ds(s * ZROWS, 128)], zsem).wait()
            return 0
        lax.fori_loop(0, ZROWS // 128, zdrain, 0)
        plsc.subcore_barrier()

        def gather(k, buf, sem):
            return pltpu.make_async_copy(
                g_hbm.at[src_v.at[pl.ds(k * CH, CH)]], buf, sem)

        def scat_start(k, q, sem):
            buf = rbufs[k % 2]
            pltpu.async_copy(buf.at[pl.ds(q * 128, 128)],
                             acc_sh.at[dst_v.at[k * 4 + q]], sem, add=True)

        def scat_wait(k, q, sem):
            buf = rbufs[k % 2]
            pltpu.make_async_copy(buf.at[pl.ds(q * 128, 128)],
                                  acc_sh.at[dst_v.at[k * 4 + q]], sem).wait()

        def sblock(sb, _):
            pltpu.sync_copy(srcf_hbm.at[pl.ds(e0 + sb * 4 * CH, 4 * CH)],
                            src_v)
            pltpu.sync_copy(dst_hbm.at[pl.ds(e0 // 128 + sb * SB, SB)], dst_v)
            gather(0, rb0_v, gsems.at[0]).start()
            gather(1, rb1_v, gsems.at[1]).start()
            for k in range(4):
                sl = k % 2
                gather(k, rbufs[sl], gsems.at[sl]).wait()
                for q in range(4):
                    scat_start(k, q, ssems.at[sl])
                if k + 2 < 4:
                    for q in range(4):
                        scat_wait(k, q, ssems.at[sl])
                    gather(k + 2, rbufs[sl], gsems.at[sl]).start()
            for k in (2, 3):
                for q in range(4):
                    scat_wait(k, q, ssems.at[k % 2])
            return 0
        lax.fori_loop(0, nsb, sblock, 0)
        plsc.subcore_barrier()
        pltpu.sync_copy(acc_sh.at[pl.ds(s * ZROWS, ZROWS)],
                        out_hbm.at[c, pl.ds(s * ZROWS, ZROWS)])
        plsc.subcore_barrier()

    run_half(glo_hbm, outlo_hbm)
    run_half(ghi_hbm, outhi_hbm)


@functools.partial(
    pl.kernel,
    out_type=jax.ShapeDtypeStruct((N_PAIRS,), _f32),
    mesh=_MESH,
    compiler_params=pltpu.CompilerParams(use_tc_tiling_on_sc=False, needs_layout_passes=False),
    scratch_types=[
        pltpu.VMEM((PR_W, 128), _i32),
        pltpu.VMEM((PR_W, 128), _i32),
        pltpu.VMEM((2, 128, HALF), _f32),
        pltpu.VMEM((2, 128, HALF), _f32),
        pltpu.VMEM((2, 128, HALF), _f32),
        pltpu.VMEM((2, 128, HALF), _f32),
        pltpu.VMEM((PR_W * 128,), _f32),
        pltpu.SemaphoreType.DMA((2,)),
        pltpu.SemaphoreType.DMA((2,)),
    ],
)
def _decode_kernel(zlo_hbm, zhi_hbm, u_hbm, v_hbm, scores_hbm,
                   u_v, v_v, ul_v, uh_v, vl_v, vh_v, sc_v, usems, vsems):
    c = lax.axis_index("c")
    s = lax.axis_index("s")
    w = c * 16 + s
    pltpu.sync_copy(u_hbm.at[pl.ds(w * PR_W, PR_W)], u_v)
    pltpu.sync_copy(v_hbm.at[pl.ds(w * PR_W, PR_W)], v_v)
    iota16 = lax.iota(_i32, 16)

    def fetch(j, sl):
        return (pltpu.make_async_copy(zlo_hbm.at[u_v.at[j]], ul_v.at[sl],
                                      usems.at[sl]),
                pltpu.make_async_copy(zhi_hbm.at[u_v.at[j]], uh_v.at[sl],
                                      usems.at[sl]),
                pltpu.make_async_copy(zlo_hbm.at[v_v.at[j]], vl_v.at[sl],
                                      vsems.at[sl]),
                pltpu.make_async_copy(zhi_hbm.at[v_v.at[j]], vh_v.at[sl],
                                      vsems.at[sl]))

    def compute(j, sl):
        ul, uh, vl, vh = (ul_v.at[sl], uh_v.at[sl], vl_v.at[sl], vh_v.at[sl])

        def grp(g, _):
            ridx = g * 16 + iota16
            sc = jnp.zeros((16,), _f32)
            for jf in range(HALF):
                cidx = jnp.full((16,), jf, _i32)
                sc = sc + (plsc.load_gather(ul, [ridx, cidx]) *
                           plsc.load_gather(vl, [ridx, cidx]))
                sc = sc + (plsc.load_gather(uh, [ridx, cidx]) *
                           plsc.load_gather(vh, [ridx, cidx]))
            sc_v[pl.ds(j * 128 + g * 16, 16)] = sc
            return 0
        lax.fori_loop(0, 8, grp, 0)

    for desc in fetch(0, 0):
        desc.start()
    for desc in fetch(1, 1):
        desc.start()

    def chunk(jp, _):
        j = jp * 2
        for desc in fetch(j, 0):
            desc.wait()
        compute(j, 0)

        @pl.when(j + 2 < PR_W)
        def _():
            for desc in fetch(j + 2, 0):
                desc.start()

        for desc in fetch(j + 1, 1):
            desc.wait()
        compute(j + 1, 1)

        @pl.when(j + 3 < PR_W)
        def _():
            for desc in fetch(j + 3, 1):
                desc.start()
        return 0
    lax.fori_loop(0, PR_W // 2, chunk, 0)
    pltpu.sync_copy(sc_v, scores_hbm.at[pl.ds(w * PR_W * 128, PR_W * 128)])


_BLK = 2048
_WBLK = _BLK * HALF // 128
_GRID = P_NODES // _BLK


def _wide_mask(i):
    rows = lax.broadcasted_iota(_i32, (_WBLK, 1), 0) * 8 + i * _BLK
    lanes = lax.broadcasted_iota(_i32, (1, 128), 1) // HALF
    return (rows + lanes) < N_NODES


def _tc_prep1(degp_ref, embg_ref, w1lo_ref, w1hi_ref,
              dinv_ref, glo_ref, ghi_ref):
    i = pl.program_id(0)
    deg = degp_ref[0] + degp_ref[1] + 1.0
    dinv = lax.rsqrt(deg)
    dinv_ref[...] = dinv
    eg = embg_ref[...]
    hlo = jnp.dot(eg, w1lo_ref[...], preferred_element_type=_f32)
    hhi = jnp.dot(eg, w1hi_ref[...], preferred_element_type=_f32)
    m = _wide_mask(i)
    glo_ref[...] = jnp.where(m, hlo * dinv, 0.0)
    ghi_ref[...] = jnp.where(m, hhi * dinv, 0.0)


def _tc_mid(alo_ref, ahi_ref, glo_ref, ghi_ref, dinv_ref, b1lo_ref, b1hi_ref,
            wll_ref, whl_ref, wlh_ref, whh_ref, g2lo_ref, g2hi_ref):
    i = pl.program_id(0)
    dinv = dinv_ref[...]
    z1lo = jax.nn.relu(
        dinv * (alo_ref[0] + alo_ref[1] + glo_ref[...]) + b1lo_ref[...])
    z1hi = jax.nn.relu(
        dinv * (ahi_ref[0] + ahi_ref[1] + ghi_ref[...]) + b1hi_ref[...])
    hlo = (jnp.dot(z1lo, wll_ref[...], preferred_element_type=_f32) +
           jnp.dot(z1hi, whl_ref[...], preferred_element_type=_f32))
    hhi = (jnp.dot(z1lo, wlh_ref[...], preferred_element_type=_f32) +
           jnp.dot(z1hi, whh_ref[...], preferred_element_type=_f32))
    m = _wide_mask(i)
    g2lo_ref[...] = jnp.where(m, hlo * dinv, 0.0)
    g2hi_ref[...] = jnp.where(m, hhi * dinv, 0.0)


def _tc_final(alo_ref, ahi_ref, glo_ref, ghi_ref, dinv_ref, b2lo_ref,
              b2hi_ref, zlo_ref, zhi_ref):
    dinv = dinv_ref[...]
    zlo_ref[...] = dinv * (alo_ref[0] + alo_ref[1] + glo_ref[...]) + b2lo_ref[...]
    zhi_ref[...] = dinv * (ahi_ref[0] + ahi_ref[1] + ghi_ref[...]) + b2hi_ref[...]


def _bs_wide():
    return pl.BlockSpec((_WBLK, 128), lambda i: (i, 0))


def _bs_part():
    return pl.BlockSpec((2, _WBLK, 128), lambda i: (0, i, 0))


def _bs_bias():
    return pl.BlockSpec((1, 128), lambda i: (0, 0))


_WSHAPE = jax.ShapeDtypeStruct((NB, 128), _f32)


def _prep1(degp, embg, W1lo, W1hi):
    return pl.pallas_call(
        _tc_prep1,
        grid=(_GRID,),
        in_specs=[_bs_part(),
                  pl.BlockSpec((_WBLK, 256), lambda i: (i, 0)),
                  pl.BlockSpec((256, 128), lambda i: (0, 0)),
                  pl.BlockSpec((256, 128), lambda i: (0, 0))],
        out_specs=[_bs_wide(), _bs_wide(), _bs_wide()],
        out_shape=[_WSHAPE] * 3,
    )(degp, embg, W1lo, W1hi)


def _mid(alo, ahi, glo, ghi, dinv, b1lo, b1hi, wbd):
    return pl.pallas_call(
        _tc_mid,
        grid=(_GRID,),
        in_specs=[_bs_part(), _bs_part(), _bs_wide(), _bs_wide(), _bs_wide(),
                  _bs_bias(), _bs_bias()] +
                 [pl.BlockSpec((128, 128), lambda i: (0, 0))] * 4,
        out_specs=[_bs_wide(), _bs_wide()],
        out_shape=[_WSHAPE] * 2,
    )(alo, ahi, glo, ghi, dinv, b1lo, b1hi, *wbd)


def _final(alo, ahi, glo, ghi, dinv, b2lo, b2hi):
    return pl.pallas_call(
        _tc_final,
        grid=(_GRID,),
        in_specs=[_bs_part(), _bs_part(), _bs_wide(), _bs_wide(), _bs_wide(),
                  _bs_bias(), _bs_bias()],
        out_specs=[_bs_wide(), _bs_wide()],
        out_shape=[_WSHAPE] * 2,
    )(alo, ahi, glo, ghi, dinv, b2lo, b2hi)


def _as_half(x_wide):
    return x_wide.reshape(P_NODES, HALF)


def _as_wide2(x_part):
    return x_part.reshape(2, NB, 128)


def kernel(edge_index, edge_pairs, emb, W1, b1, W2, b2):
    pad = N_NODES + jnp.arange(EP - N_EDGES, dtype=_i32) % (P_NODES - N_NODES)
    srcf = jnp.concatenate([edge_index[0], pad])
    dst2d = jnp.concatenate([edge_index[1], pad]).reshape(EROWS, 128)
    u2d = edge_pairs[0].reshape(PROWS, 128)
    v2d = edge_pairs[1].reshape(PROWS, 128)
    b1lo = jnp.tile(b1[:HALF], 8).reshape(1, 128)
    b1hi = jnp.tile(b1[HALF:], 8).reshape(1, 128)
    b2lo = jnp.tile(b2[:HALF], 8).reshape(1, 128)
    b2hi = jnp.tile(b2[HALF:], 8).reshape(1, 128)
    eye8 = jnp.eye(8, dtype=_f32)
    W1lo = jnp.kron(eye8, W1[:, :HALF])
    W1hi = jnp.kron(eye8, W1[:, HALF:])
    wbd = (jnp.kron(eye8, W2[:HALF, :HALF]), jnp.kron(eye8, W2[HALF:, :HALF]),
           jnp.kron(eye8, W2[:HALF, HALF:]), jnp.kron(eye8, W2[HALF:, HALF:]))
    embg = emb.reshape(N_NODES * DIM // 256, 256)

    degp = _hist_kernel(dst2d)
    dinv, g1lo, g1hi = _prep1(_as_wide2(degp), embg, W1lo, W1hi)
    a1lo, a1hi = _scatter_kernel(srcf, dst2d, _as_half(g1lo), _as_half(g1hi))
    g2lo, g2hi = _mid(_as_wide2(a1lo), _as_wide2(a1hi), g1lo, g1hi, dinv,
                      b1lo, b1hi, wbd)
    a2lo, a2hi = _scatter_kernel(srcf, dst2d, _as_half(g2lo), _as_half(g2hi))
    zlo, zhi = _final(_as_wide2(a2lo), _as_wide2(a2hi), g2lo, g2hi, dinv,
                      b2lo, b2hi)
    return _decode_kernel(_as_half(zlo), _as_half(zhi), u2d, v2d)


# speedup vs baseline: 48.6371x; 1.0364x over previous
import functools

import jax
import jax.numpy as jnp
from jax import lax
from jax.experimental import pallas as pl
from jax.experimental.pallas import tpu as pltpu
from jax.experimental.pallas import tpu_sc as plsc

N_NODES = 100000
P_NODES = 100352
DIM = 32
HALF = 16
N_EDGES = 1600000
EP = 1638400
EROWS = EP // 128
NW = 32
RW = EROWS // NW
SB = 16
NSB = RW // SB
CH = 512
EW0 = 51200
EW1 = 51200
E0 = EW0 * 16
NSB0 = EW0 // (4 * CH)
NSB1 = EW1 // (4 * CH)
RW0 = EW0 // 128
RW1 = EW1 // 128
ZROWS = P_NODES // 16
N_PAIRS = 262144
PROWS = N_PAIRS // 128
PR_W = PROWS // NW
NB = P_NODES * HALF // 128

_f32 = jnp.float32
_i32 = jnp.int32

_MESH = plsc.VectorSubcoreMesh(
    core_axis_name="c", subcore_axis_name="s", num_cores=2, num_subcores=16)


def _fill_rows(ref, nrows, value):
    def body(i, _):
        ref[i] = jnp.full((16,), value, _f32)
        return 0
    lax.fori_loop(0, nrows, body, 0)


@functools.partial(
    pl.kernel,
    out_type=jax.ShapeDtypeStruct((2, P_NODES, HALF), _f32),
    mesh=_MESH,
    compiler_params=pltpu.CompilerParams(use_tc_tiling_on_sc=False, needs_layout_passes=False),
    scratch_types=[
        pltpu.VMEM((SB, 128), _i32),
        pltpu.VMEM((SB, 128), _i32),
        pltpu.VMEM((128, HALF), _f32),
        pltpu.VMEM((128, HALF), _f32),
        pltpu.VMEM_SHARED((P_NODES, HALF), _f32),
        pltpu.SemaphoreType.DMA((2,)),
        pltpu.SemaphoreType.DMA,
    ],
)
def _hist_kernel(dst_hbm, out_hbm, dst0_v, dst1_v, ones_v, zero_v, acc_sh,
                 ssems, zsem):
    c = lax.axis_index("c")
    s = lax.axis_index("s")
    row0 = jnp.where(c == 0, s * RW0, E0 // 128 + s * RW1)
    nsb = jnp.where(c == 0, RW0 // SB, RW1 // SB)
    _fill_rows(ones_v, 128, 1.0)
    _fill_rows(zero_v, 128, 0.0)
    dbufs = (dst0_v, dst1_v)

    def zfire(i, _):
        pltpu.make_async_copy(
            zero_v, acc_sh.at[pl.ds(s * ZROWS + i * 128, 128)], zsem).start()
        return 0
    lax.fori_loop(0, ZROWS // 128, zfire, 0)

    def zdrain(i, _):
        pltpu.make_async_copy(
            zero_v, acc_sh.at[pl.ds(s * ZROWS, 128)], zsem).wait()
        return 0
    lax.fori_loop(0, ZROWS // 128, zdrain, 0)
    plsc.subcore_barrier()

    def stage(sb, buf):
        pltpu.sync_copy(dst_hbm.at[pl.ds(row0 + sb * SB, SB)], buf)

    def fire(buf, sem):
        for j in range(SB):
            pltpu.async_copy(ones_v, acc_sh.at[buf.at[j]], sem, add=True)

    def drain(buf, sem):
        for j in range(SB):
            pltpu.make_async_copy(ones_v, acc_sh.at[buf.at[j]], sem).wait()

    stage(0, dst0_v)
    fire(dst0_v, ssems.at[0])

    def sblock(p, _):
        sb = p * 2

        @pl.when(sb + 1 < nsb)
        def _():
            stage(sb + 1, dst1_v)
            fire(dst1_v, ssems.at[1])
        drain(dst0_v, ssems.at[0])

        @pl.when(sb + 2 < nsb)
        def _():
            stage(sb + 2, dst0_v)
            fire(dst0_v, ssems.at[0])

        @pl.when(sb + 1 < nsb)
        def _():
            drain(dst1_v, ssems.at[1])
        return 0
    lax.fori_loop(0, (nsb + 1) // 2, sblock, 0)
    plsc.subcore_barrier()
    pltpu.sync_copy(acc_sh.at[pl.ds(s * ZROWS, ZROWS)],
                    out_hbm.at[c, pl.ds(s * ZROWS, ZROWS)])


@functools.partial(
    pl.kernel,
    out_type=(jax.ShapeDtypeStruct((2, P_NODES, HALF), _f32),
              jax.ShapeDtypeStruct((2, P_NODES, HALF), _f32)),
    mesh=_MESH,
    compiler_params=pltpu.CompilerParams(use_tc_tiling_on_sc=False, needs_layout_passes=False),
    scratch_types=[
        pltpu.VMEM((2048,), _i32),
        pltpu.VMEM((SB, 128), _i32),
        pltpu.VMEM((4, 256, HALF), _f32),
        pltpu.VMEM((128, HALF), _f32),
        pltpu.VMEM_SHARED((P_NODES, HALF), _f32),
        pltpu.SemaphoreType.DMA((4,)),
        pltpu.SemaphoreType.DMA((4,)),
        pltpu.SemaphoreType.DMA,
    ],
)
def _scatter_kernel(srcf_hbm, dst_hbm, glo_hbm, ghi_hbm, outlo_hbm, outhi_hbm,
                    src_v, dst_v, rb_v, zero_v, acc_sh,
                    gsems, ssems, zsem):
    c = lax.axis_index("c")
    s = lax.axis_index("s")
    e0 = jnp.where(c == 0, s * EW0, E0 + s * EW1)
    nsb = jnp.where(c == 0, NSB0, NSB1)
    _fill_rows(zero_v, 128, 0.0)

    def run_half(g_hbm, out_hbm):
        def zfire(i, _):
            pltpu.make_async_copy(
                zero_v, acc_sh.at[pl.ds(s * ZROWS + i * 128, 128)],
                zsem).start()
            return 0
        lax.fori_loop(0, ZROWS // 128, zfire, 0)

        def zdrain(i, _):
            pltpu.make_async_copy(
                zero_v, acc_sh.at[pl.ds(s * ZROWS, 128)], zsem).wait()
            return 0
        lax.fori_loop(0, ZROWS // 128, zdrain, 0)
        plsc.subcore_barrier()

        def gather(k):
            b = k % 4
            return pltpu.make_async_copy(
                g_hbm.at[src_v.at[pl.ds(k * 256, 256)]], rb_v.at[b],
                gsems.at[b])

        def scat_start(k, q):
            b = k % 4
            pltpu.async_copy(rb_v.at[b].at[pl.ds(q * 128, 128)],
                             acc_sh.at[dst_v.at[k * 2 + q]], ssems.at[b],
                             add=True)

        def scat_wait(k, q):
            b = k % 4
            pltpu.make_async_copy(rb_v.at[b].at[pl.ds(q * 128, 128)],
                                  acc_sh.at[dst_v.at[k * 2 + q]],
                                  ssems.at[b]).wait()

        def sblock(sb, _):
            pltpu.sync_copy(srcf_hbm.at[pl.ds(e0 + sb * 2048, 2048)], src_v)
            pltpu.sync_copy(dst_hbm.at[pl.ds(e0 // 128 + sb * SB, SB)], dst_v)
            for k in range(4):
                gather(k).start()
            for k in range(8):
                gather(k).wait()
                scat_start(k, 0)
                scat_start(k, 1)
                if k + 4 < 8:
                    scat_wait(k, 0)
                    scat_wait(k, 1)
                    gather(k + 4).start()
            for k in range(4, 8):
                scat_wait(k, 0)
                scat_wait(k, 1)
            return 0
        lax.fori_loop(0, nsb, sblock, 0)
        plsc.subcore_barrier()
        pltpu.sync_copy(acc_sh.at[pl.ds(s * ZROWS, ZROWS)],
                        out_hbm.at[c, pl.ds(s * ZROWS, ZROWS)])
        plsc.subcore_barrier()

    run_half(glo_hbm, outlo_hbm)
    run_half(ghi_hbm, outhi_hbm)


@functools.partial(
    pl.kernel,
    out_type=jax.ShapeDtypeStruct((N_PAIRS,), _f32),
    mesh=_MESH,
    compiler_params=pltpu.CompilerParams(use_tc_tiling_on_sc=False, needs_layout_passes=False),
    scratch_types=[
        pltpu.VMEM((PR_W, 128), _i32),
        pltpu.VMEM((PR_W, 128), _i32),
        pltpu.VMEM((2, 128, HALF), _f32),
        pltpu.VMEM((2, 128, HALF), _f32),
        pltpu.VMEM((2, 128, HALF), _f32),
        pltpu.VMEM((2, 128, HALF), _f32),
        pltpu.VMEM((PR_W * 128,), _f32),
        pltpu.SemaphoreType.DMA((2,)),
        pltpu.SemaphoreType.DMA((2,)),
    ],
)
def _decode_kernel(zlo_hbm, zhi_hbm, u_hbm, v_hbm, scores_hbm,
                   u_v, v_v, ul_v, uh_v, vl_v, vh_v, sc_v, usems, vsems):
    c = lax.axis_index("c")
    s = lax.axis_index("s")
    w = c * 16 + s
    pltpu.sync_copy(u_hbm.at[pl.ds(w * PR_W, PR_W)], u_v)
    pltpu.sync_copy(v_hbm.at[pl.ds(w * PR_W, PR_W)], v_v)
    iota16 = lax.iota(_i32, 16)

    def fetch(j, sl):
        return (pltpu.make_async_copy(zlo_hbm.at[u_v.at[j]], ul_v.at[sl],
                                      usems.at[sl]),
                pltpu.make_async_copy(zhi_hbm.at[u_v.at[j]], uh_v.at[sl],
                                      usems.at[sl]),
                pltpu.make_async_copy(zlo_hbm.at[v_v.at[j]], vl_v.at[sl],
                                      vsems.at[sl]),
                pltpu.make_async_copy(zhi_hbm.at[v_v.at[j]], vh_v.at[sl],
                                      vsems.at[sl]))

    def compute(j, sl):
        ul, uh, vl, vh = (ul_v.at[sl], uh_v.at[sl], vl_v.at[sl], vh_v.at[sl])

        def grp(g, _):
            ridx = g * 16 + iota16
            sc = jnp.zeros((16,), _f32)
            for jf in range(HALF):
                cidx = jnp.full((16,), jf, _i32)
                sc = sc + (plsc.load_gather(ul, [ridx, cidx]) *
                           plsc.load_gather(vl, [ridx, cidx]))
                sc = sc + (plsc.load_gather(uh, [ridx, cidx]) *
                           plsc.load_gather(vh, [ridx, cidx]))
            sc_v[pl.ds(j * 128 + g * 16, 16)] = sc
            return 0
        lax.fori_loop(0, 8, grp, 0)

    for desc in fetch(0, 0):
        desc.start()
    for desc in fetch(1, 1):
        desc.start()

    def chunk(jp, _):
        j = jp * 2
        for desc in fetch(j, 0):
            desc.wait()
        compute(j, 0)

        @pl.when(j + 2 < PR_W)
        def _():
            for desc in fetch(j + 2, 0):
                desc.start()

        for desc in fetch(j + 1, 1):
            desc.wait()
        compute(j + 1, 1)

        @pl.when(j + 3 < PR_W)
        def _():
            for desc in fetch(j + 3, 1):
                desc.start()
        return 0
    lax.fori_loop(0, PR_W // 2, chunk, 0)
    pltpu.sync_copy(sc_v, scores_hbm.at[pl.ds(w * PR_W * 128, PR_W * 128)])


_BLK = 2048
_WBLK = _BLK * HALF // 128
_GRID = P_NODES // _BLK


def _wide_mask(i):
    rows = lax.broadcasted_iota(_i32, (_WBLK, 1), 0) * 8 + i * _BLK
    lanes = lax.broadcasted_iota(_i32, (1, 128), 1) // HALF
    return (rows + lanes) < N_NODES


def _tc_prep1(degp_ref, embg_ref, w1lo_ref, w1hi_ref,
              dinv_ref, glo_ref, ghi_ref):
    i = pl.program_id(0)
    deg = degp_ref[0] + degp_ref[1] + 1.0
    dinv = lax.rsqrt(deg)
    dinv_ref[...] = dinv
    eg = embg_ref[...]
    hlo = jnp.dot(eg, w1lo_ref[...], preferred_element_type=_f32)
    hhi = jnp.dot(eg, w1hi_ref[...], preferred_element_type=_f32)
    m = _wide_mask(i)
    glo_ref[...] = jnp.where(m, hlo * dinv, 0.0)
    ghi_ref[...] = jnp.where(m, hhi * dinv, 0.0)


def _tc_mid(alo_ref, ahi_ref, glo_ref, ghi_ref, dinv_ref, b1lo_ref, b1hi_ref,
            wll_ref, whl_ref, wlh_ref, whh_ref, g2lo_ref, g2hi_ref):
    i = pl.program_id(0)
    dinv = dinv_ref[...]
    z1lo = jax.nn.relu(
        dinv * (alo_ref[0] + alo_ref[1] + glo_ref[...]) + b1lo_ref[...])
    z1hi = jax.nn.relu(
        dinv * (ahi_ref[0] + ahi_ref[1] + ghi_ref[...]) + b1hi_ref[...])
    hlo = (jnp.dot(z1lo, wll_ref[...], preferred_element_type=_f32) +
           jnp.dot(z1hi, whl_ref[...], preferred_element_type=_f32))
    hhi = (jnp.dot(z1lo, wlh_ref[...], preferred_element_type=_f32) +
           jnp.dot(z1hi, whh_ref[...], preferred_element_type=_f32))
    m = _wide_mask(i)
    g2lo_ref[...] = jnp.where(m, hlo * dinv, 0.0)
    g2hi_ref[...] = jnp.where(m, hhi * dinv, 0.0)


def _tc_final(alo_ref, ahi_ref, glo_ref, ghi_ref, dinv_ref, b2lo_ref,
              b2hi_ref, zlo_ref, zhi_ref):
    dinv = dinv_ref[...]
    zlo_ref[...] = dinv * (alo_ref[0] + alo_ref[1] + glo_ref[...]) + b2lo_ref[...]
    zhi_ref[...] = dinv * (ahi_ref[0] + ahi_ref[1] + ghi_ref[...]) + b2hi_ref[...]


def _bs_wide():
    return pl.BlockSpec((_WBLK, 128), lambda i: (i, 0))


def _bs_part():
    return pl.BlockSpec((2, _WBLK, 128), lambda i: (0, i, 0))


def _bs_bias():
    return pl.BlockSpec((1, 128), lambda i: (0, 0))


_WSHAPE = jax.ShapeDtypeStruct((NB, 128), _f32)


def _prep1(degp, embg, W1lo, W1hi):
    return pl.pallas_call(
        _tc_prep1,
        grid=(_GRID,),
        in_specs=[_bs_part(),
                  pl.BlockSpec((_WBLK, 256), lambda i: (i, 0)),
                  pl.BlockSpec((256, 128), lambda i: (0, 0)),
                  pl.BlockSpec((256, 128), lambda i: (0, 0))],
        out_specs=[_bs_wide(), _bs_wide(), _bs_wide()],
        out_shape=[_WSHAPE] * 3,
    )(degp, embg, W1lo, W1hi)


def _mid(alo, ahi, glo, ghi, dinv, b1lo, b1hi, wbd):
    return pl.pallas_call(
        _tc_mid,
        grid=(_GRID,),
        in_specs=[_bs_part(), _bs_part(), _bs_wide(), _bs_wide(), _bs_wide(),
                  _bs_bias(), _bs_bias()] +
                 [pl.BlockSpec((128, 128), lambda i: (0, 0))] * 4,
        out_specs=[_bs_wide(), _bs_wide()],
        out_shape=[_WSHAPE] * 2,
    )(alo, ahi, glo, ghi, dinv, b1lo, b1hi, *wbd)


def _final(alo, ahi, glo, ghi, dinv, b2lo, b2hi):
    return pl.pallas_call(
        _tc_final,
        grid=(_GRID,),
        in_specs=[_bs_part(), _bs_part(), _bs_wide(), _bs_wide(), _bs_wide(),
                  _bs_bias(), _bs_bias()],
        out_specs=[_bs_wide(), _bs_wide()],
        out_shape=[_WSHAPE] * 2,
    )(alo, ahi, glo, ghi, dinv, b2lo, b2hi)


def _as_half(x_wide):
    return x_wide.reshape(P_NODES, HALF)


def _as_wide2(x_part):
    return x_part.reshape(2, NB, 128)


def kernel(edge_index, edge_pairs, emb, W1, b1, W2, b2):
    pad = N_NODES + jnp.arange(EP - N_EDGES, dtype=_i32) % (P_NODES - N_NODES)
    srcf = jnp.concatenate([edge_index[0], pad])
    dst2d = jnp.concatenate([edge_index[1], pad]).reshape(EROWS, 128)
    u2d = edge_pairs[0].reshape(PROWS, 128)
    v2d = edge_pairs[1].reshape(PROWS, 128)
    b1lo = jnp.tile(b1[:HALF], 8).reshape(1, 128)
    b1hi = jnp.tile(b1[HALF:], 8).reshape(1, 128)
    b2lo = jnp.tile(b2[:HALF], 8).reshape(1, 128)
    b2hi = jnp.tile(b2[HALF:], 8).reshape(1, 128)
    eye8 = jnp.eye(8, dtype=_f32)
    W1lo = jnp.kron(eye8, W1[:, :HALF])
    W1hi = jnp.kron(eye8, W1[:, HALF:])
    wbd = (jnp.kron(eye8, W2[:HALF, :HALF]), jnp.kron(eye8, W2[HALF:, :HALF]),
           jnp.kron(eye8, W2[:HALF, HALF:]), jnp.kron(eye8, W2[HALF:, HALF:]))
    embg = emb.reshape(N_NODES * DIM // 256, 256)

    degp = _hist_kernel(dst2d)
    dinv, g1lo, g1hi = _prep1(_as_wide2(degp), embg, W1lo, W1hi)
    a1lo, a1hi = _scatter_kernel(srcf, dst2d, _as_half(g1lo), _as_half(g1hi))
    g2lo, g2hi = _mid(_as_wide2(a1lo), _as_wide2(a1hi), g1lo, g1hi, dinv,
                      b1lo, b1hi, wbd)
    a2lo, a2hi = _scatter_kernel(srcf, dst2d, _as_half(g2lo), _as_half(g2hi))
    zlo, zhi = _final(_as_wide2(a2lo), _as_wide2(a2hi), g2lo, g2hi, dinv,
                      b2lo, b2hi)
    return _decode_kernel(_as_half(zlo), _as_half(zhi), u2d, v2d)
